# baseline (speedup 1.0000x reference)
import functools

import jax
import jax.numpy as jnp
from jax import lax
from jax.experimental import pallas as pl
from jax.experimental.pallas import tpu as pltpu
from jax.experimental.pallas import tpu_sc as plsc

NUM_FIRMS = 50000
NUM_PRODS = 256
E = 200000
DEBT_PENALTY = 5.0
CONSUMPTION_REWARD = 4.0

NC = 2
NS = 16
NW = NC * NS
LANES = 16

RANGE_WORDS = 100352
N_PASSES = 4
TOT_ROWS = (NW * N_PASSES * RANGE_WORDS) // NUM_PRODS

CHUNK = 8000
N_CHUNKS = E // CHUNK
VECS_PER_CHUNK = CHUNK // LANES


def _scatter_kernel(src_hbm, prod_hbm, amt_hbm, tot_hbm,
                    acc, s_src, s_prod, s_amt):
    wid = lax.axis_index("s") * NC + lax.axis_index("c")

    def pass_body(p, _):
        base = (p * NW + wid) * RANGE_WORDS

        def zero_body(i, _):
            acc[pl.ds(i * LANES, LANES)] = jnp.zeros((LANES,), jnp.float32)
            return 0
        lax.fori_loop(0, RANGE_WORDS // LANES, zero_body, 0)

        def chunk_body(c, _):
            off = c * CHUNK
            pltpu.sync_copy(src_hbm.at[pl.ds(off, CHUNK)], s_src)
            pltpu.sync_copy(prod_hbm.at[pl.ds(off, CHUNK)], s_prod)
            pltpu.sync_copy(amt_hbm.at[pl.ds(off, CHUNK)], s_amt)

            def vec_body(j, _):
                sl = pl.ds(j * LANES, LANES)
                sv = s_src[sl]
                pv = s_prod[sl]
                av = s_amt[sl]
                flat = sv * NUM_PRODS + (pv - NUM_FIRMS)
                local = flat - base
                mask = (local >= 0) & (local < RANGE_WORDS)
                idx = jnp.where(mask, local, 0)
                av = jnp.maximum(av, 0.0)
                plsc.addupdate_scatter(acc, [idx], av, mask=mask)
                return 0
            lax.fori_loop(0, VECS_PER_CHUNK, vec_body, 0)
            return 0
        lax.fori_loop(0, N_CHUNKS, chunk_body, 0)

        pltpu.sync_copy(acc, tot_hbm.at[pl.ds(base, RANGE_WORDS)])
        return 0
    lax.fori_loop(0, N_PASSES, pass_body, 0)


@jax.jit
def _sc_scatter(src, prod, amt):
    mesh = plsc.VectorSubcoreMesh(core_axis_name="c", subcore_axis_name="s")
    return pl.kernel(
        _scatter_kernel,
        out_type=jax.ShapeDtypeStruct((TOT_ROWS * NUM_PRODS,), jnp.float32),
        mesh=mesh,
        compiler_params=pltpu.CompilerParams(needs_layout_passes=False),
        scratch_types=[
            pltpu.VMEM((RANGE_WORDS,), jnp.float32),
            pltpu.VMEM((CHUNK,), jnp.int32),
            pltpu.VMEM((CHUNK,), jnp.int32),
            pltpu.VMEM((CHUNK,), jnp.float32),
        ],
    )(src, prod, amt)


ROWS_PER_BLOCK = 512
N_BLOCKS = TOT_ROWS // ROWS_PER_BLOCK


def _reduce_kernel(tot_ref, aw_ref, debt_ref, cons_ref):
    i = pl.program_id(0)
    aw = aw_ref[...]
    r = lax.broadcasted_iota(jnp.int32, (NUM_PRODS, NUM_PRODS), 0)
    c = lax.broadcasted_iota(jnp.int32, (NUM_PRODS, NUM_PRODS), 1)
    att = jnp.maximum(jnp.where(r == c, 0.0, aw), 0.0)
    cons = jnp.dot(tot_ref[...], att, preferred_element_type=jnp.float32)
    d = jnp.sum(jnp.maximum(cons - 1.0, 0.0))
    s = jnp.sum(cons)

    @pl.when(i == 0)
    def _init():
        debt_ref[0, 0] = d
        cons_ref[0, 0] = s

    @pl.when(i != 0)
    def _acc():
        debt_ref[0, 0] += d
        cons_ref[0, 0] += s


@jax.jit
def _tc_reduce(totals2d, att_weights):
    return pl.pallas_call(
        _reduce_kernel,
        grid=(N_BLOCKS,),
        in_specs=[
            pl.BlockSpec((ROWS_PER_BLOCK, NUM_PRODS), lambda i: (i, 0)),
            pl.BlockSpec((NUM_PRODS, NUM_PRODS), lambda i: (0, 0)),
        ],
        out_specs=[
            pl.BlockSpec(memory_space=pltpu.SMEM),
            pl.BlockSpec(memory_space=pltpu.SMEM),
        ],
        out_shape=[
            jax.ShapeDtypeStruct((1, 1), jnp.float32),
            jax.ShapeDtypeStruct((1, 1), jnp.float32),
        ],
    )(totals2d, att_weights)


def kernel(src, dst, prod, t, amt, att_weights):
    totals = _sc_scatter(src, prod, amt)
    totals2d = totals.reshape(TOT_ROWS, NUM_PRODS)
    debt_sum, cons_sum = _tc_reduce(totals2d, att_weights)
    n = src.shape[0]
    debt_loss = DEBT_PENALTY * debt_sum[0, 0] / NUM_FIRMS
    consump_rwd = CONSUMPTION_REWARD * cons_sum[0, 0] / NUM_FIRMS
    inv_loss = debt_loss - consump_rwd
    return (inv_loss / n, debt_loss / n, consump_rwd / n)


# speedup vs baseline: 1.6130x; 1.6130x over previous
import jax
import jax.numpy as jnp
from jax import lax
from jax.experimental import pallas as pl
from jax.experimental.pallas import tpu as pltpu
from jax.experimental.pallas import tpu_sc as plsc

NUM_FIRMS = 50000
NUM_PRODS = 256
E = 200000
DEBT_PENALTY = 5.0
CONSUMPTION_REWARD = 4.0

NC = 2
NS = 16
NW = NC * NS
LANES = 16

RANGE_WORDS = 100352
N_PASSES = 4
TOT_ROWS = (NW * N_PASSES * RANGE_WORDS) // NUM_PRODS

E_PAD = 200704

PREP_PER_TILE = E_PAD // NS
PREP_SUB = 1568
N_PREP_SUB = PREP_PER_TILE // PREP_SUB

SCAN_CHUNK = 3584
N_SCAN_CHUNKS = E_PAD // SCAN_CHUNK
N_SCAN_PAIRS = N_SCAN_CHUNKS // 2


def _scatter_kernel(src_hbm, prod_hbm, amt_hbm,
                    tot_hbm, flat_h, amtc_h, acc,
                    p_src, p_prod, p_amt, p_flat, p_amtc,
                    bf0, ba0, bf1, ba1,
                    sf0, sa0, sf1, sa1):
    s = lax.axis_index("s")
    c = lax.axis_index("c")
    wid = s * NC + c

    tile_base = s * PREP_PER_TILE

    def prep_sub(k, _):
        off = tile_base + k * PREP_SUB
        pltpu.sync_copy(src_hbm.at[pl.ds(off, PREP_SUB)], p_src)
        pltpu.sync_copy(prod_hbm.at[pl.ds(off, PREP_SUB)], p_prod)
        pltpu.sync_copy(amt_hbm.at[pl.ds(off, PREP_SUB)], p_amt)

        def vb(j, _):
            sl = pl.ds(j * LANES, LANES)
            p_flat[sl] = p_src[sl] * NUM_PRODS + (p_prod[sl] - NUM_FIRMS)
            p_amtc[sl] = jnp.maximum(p_amt[sl], 0.0)
            return 0
        lax.fori_loop(0, PREP_SUB // LANES, vb, 0)
        pltpu.sync_copy(p_flat, flat_h.at[pl.ds(c * E_PAD + off, PREP_SUB)])
        pltpu.sync_copy(p_amtc, amtc_h.at[pl.ds(c * E_PAD + off, PREP_SUB)])
        return 0
    lax.fori_loop(0, N_PREP_SUB, prep_sub, 0)
    plsc.subcore_barrier()
    core_off = c * E_PAD

    zeros16 = jnp.zeros((LANES,), jnp.float32)

    def process(bf, ba, base):
        def vb(j, _):
            for u in range(4):
                sl = pl.ds((j * 4 + u) * LANES, LANES)
                local = bf[sl] - base
                mask = local.astype(jnp.uint32) < jnp.uint32(RANGE_WORDS)
                idx = jnp.where(mask, local, 0)
                plsc.addupdate_scatter(acc, [idx], ba[sl], mask=mask)
            return 0
        lax.fori_loop(0, SCAN_CHUNK // (4 * LANES), vb, 0)

    def pass_body(p, _):
        base = (p * NW + wid) * RANGE_WORDS

        def zb(i, _):
            for u in range(4):
                acc[pl.ds((i * 4 + u) * LANES, LANES)] = zeros16
            return 0
        lax.fori_loop(0, RANGE_WORDS // (4 * LANES), zb, 0)

        pltpu.async_copy(flat_h.at[pl.ds(core_off, SCAN_CHUNK)], bf0, sf0)
        pltpu.async_copy(amtc_h.at[pl.ds(core_off, SCAN_CHUNK)], ba0, sa0)
        pltpu.async_copy(flat_h.at[pl.ds(core_off + SCAN_CHUNK, SCAN_CHUNK)], bf1, sf1)
        pltpu.async_copy(amtc_h.at[pl.ds(core_off + SCAN_CHUNK, SCAN_CHUNK)], ba1, sa1)

        def pair_body(g, _):
            pltpu.make_async_copy(flat_h.at[pl.ds(core_off, SCAN_CHUNK)], bf0, sf0).wait()
            pltpu.make_async_copy(amtc_h.at[pl.ds(core_off, SCAN_CHUNK)], ba0, sa0).wait()
            process(bf0, ba0, base)

            @pl.when(g < N_SCAN_PAIRS - 1)
            def _start0():
                off = (g * 2 + 2) * SCAN_CHUNK
                pltpu.async_copy(flat_h.at[pl.ds(core_off + off, SCAN_CHUNK)], bf0, sf0)
                pltpu.async_copy(amtc_h.at[pl.ds(core_off + off, SCAN_CHUNK)], ba0, sa0)

            pltpu.make_async_copy(flat_h.at[pl.ds(core_off, SCAN_CHUNK)], bf1, sf1).wait()
            pltpu.make_async_copy(amtc_h.at[pl.ds(core_off, SCAN_CHUNK)], ba1, sa1).wait()
            process(bf1, ba1, base)

            @pl.when(g < N_SCAN_PAIRS - 1)
            def _start1():
                off = (g * 2 + 3) * SCAN_CHUNK
                pltpu.async_copy(flat_h.at[pl.ds(core_off + off, SCAN_CHUNK)], bf1, sf1)
                pltpu.async_copy(amtc_h.at[pl.ds(core_off + off, SCAN_CHUNK)], ba1, sa1)
            return 0
        lax.fori_loop(0, N_SCAN_PAIRS, pair_body, 0)

        pltpu.sync_copy(acc, tot_hbm.at[pl.ds(base, RANGE_WORDS)])
        return 0
    lax.fori_loop(0, N_PASSES, pass_body, 0)


@jax.jit
def _sc_scatter(src, prod, amt):
    mesh = plsc.VectorSubcoreMesh(core_axis_name="c", subcore_axis_name="s")
    return pl.kernel(
        _scatter_kernel,
        out_type=(
            jax.ShapeDtypeStruct((TOT_ROWS * NUM_PRODS,), jnp.float32),
            jax.ShapeDtypeStruct((NC * E_PAD,), jnp.int32),
            jax.ShapeDtypeStruct((NC * E_PAD,), jnp.float32),
        ),
        mesh=mesh,
        compiler_params=pltpu.CompilerParams(needs_layout_passes=False),
        scratch_types=[
            pltpu.VMEM((RANGE_WORDS,), jnp.float32),
            pltpu.VMEM((PREP_SUB,), jnp.int32),
            pltpu.VMEM((PREP_SUB,), jnp.int32),
            pltpu.VMEM((PREP_SUB,), jnp.float32),
            pltpu.VMEM((PREP_SUB,), jnp.int32),
            pltpu.VMEM((PREP_SUB,), jnp.float32),
            pltpu.VMEM((SCAN_CHUNK,), jnp.int32),
            pltpu.VMEM((SCAN_CHUNK,), jnp.float32),
            pltpu.VMEM((SCAN_CHUNK,), jnp.int32),
            pltpu.VMEM((SCAN_CHUNK,), jnp.float32),
            pltpu.SemaphoreType.DMA,
            pltpu.SemaphoreType.DMA,
            pltpu.SemaphoreType.DMA,
            pltpu.SemaphoreType.DMA,
        ],
    )(src, prod, amt)


ROWS_PER_BLOCK = 512
N_BLOCKS = TOT_ROWS // ROWS_PER_BLOCK


def _reduce_kernel(tot_ref, aw_ref, debt_ref, cons_ref):
    i = pl.program_id(0)
    aw = aw_ref[...]
    r = lax.broadcasted_iota(jnp.int32, (NUM_PRODS, NUM_PRODS), 0)
    c = lax.broadcasted_iota(jnp.int32, (NUM_PRODS, NUM_PRODS), 1)
    att = jnp.maximum(jnp.where(r == c, 0.0, aw), 0.0)
    cons = jnp.dot(tot_ref[...], att, preferred_element_type=jnp.float32)
    d = jnp.sum(jnp.maximum(cons - 1.0, 0.0))
    s = jnp.sum(cons)

    @pl.when(i == 0)
    def _init():
        debt_ref[0, 0] = d
        cons_ref[0, 0] = s

    @pl.when(i != 0)
    def _acc():
        debt_ref[0, 0] += d
        cons_ref[0, 0] += s


@jax.jit
def _tc_reduce(totals2d, att_weights):
    return pl.pallas_call(
        _reduce_kernel,
        grid=(N_BLOCKS,),
        in_specs=[
            pl.BlockSpec((ROWS_PER_BLOCK, NUM_PRODS), lambda i: (i, 0)),
            pl.BlockSpec((NUM_PRODS, NUM_PRODS), lambda i: (0, 0)),
        ],
        out_specs=[
            pl.BlockSpec(memory_space=pltpu.SMEM),
            pl.BlockSpec(memory_space=pltpu.SMEM),
        ],
        out_shape=[
            jax.ShapeDtypeStruct((1, 1), jnp.float32),
            jax.ShapeDtypeStruct((1, 1), jnp.float32),
        ],
    )(totals2d, att_weights)


def kernel(src, dst, prod, t, amt, att_weights):
    pad = E_PAD - src.shape[0]
    srcp = jnp.pad(src, (0, pad))
    prodp = jnp.pad(prod, (0, pad))
    amtp = jnp.pad(amt, (0, pad))
    totals, _fh, _ah = _sc_scatter(srcp, prodp, amtp)
    totals2d = totals.reshape(TOT_ROWS, NUM_PRODS)
    debt_sum, cons_sum = _tc_reduce(totals2d, att_weights)
    n = src.shape[0]
    debt_loss = DEBT_PENALTY * debt_sum[0, 0] / NUM_FIRMS
    consump_rwd = CONSUMPTION_REWARD * cons_sum[0, 0] / NUM_FIRMS
    inv_loss = debt_loss - consump_rwd
    return (inv_loss / n, debt_loss / n, consump_rwd / n)


# speedup vs baseline: 3.7117x; 2.3011x over previous
import jax
import jax.numpy as jnp
from jax import lax
from jax.experimental import pallas as pl
from jax.experimental.pallas import tpu as pltpu
from jax.experimental.pallas import tpu_sc as plsc

NUM_FIRMS = 50000
NUM_PRODS = 256
E = 200000
DEBT_PENALTY = 5.0
CONSUMPTION_REWARD = 4.0

NC = 2
NS = 16
NW = NC * NS
LANES = 16

RANGE_WORDS = 65536
N_RANGES = 196
N_PASSES = 7
TOT_ROWS = (N_RANGES * RANGE_WORDS) // NUM_PRODS

E_PAD = 200704
PREP_PER_TILE = E_PAD // NS
PREP_SUB = 1568
N_PREP_SUB = PREP_PER_TILE // PREP_SUB

SORT_CAP = 14336
SEG_CHUNK = 512
REG_TOTAL = NC * NS * SORT_CAP + 2 * SEG_CHUNK
NBKT = 256


def _scatter_kernel(src_hbm, prod_hbm, amt_hbm,
                    tot_hbm, flat_h, amtc_h,
                    acc, raw_flat, raw_amt, sorted_flat, sorted_amt,
                    p_src, p_prod, p_amt,
                    cnt, cnt8, off_start, off_run, tbl_sh,
                    sflat, samt):
    s = lax.axis_index("s")
    c = lax.axis_index("c")
    wid = s * NC + c
    ones_i = jnp.ones((LANES,), jnp.int32)
    zeros_i = jnp.zeros((LANES,), jnp.int32)
    zeros_f = jnp.zeros((LANES,), jnp.float32)
    neg1 = jnp.full((LANES,), -1, jnp.int32)
    iota16 = lax.iota(jnp.int32, LANES)

    for i in range(NBKT // LANES):
        cnt[pl.ds(i * LANES, LANES)] = zeros_i

    def prep_sub(k, _):
        off = s * PREP_PER_TILE + k * PREP_SUB
        pltpu.sync_copy(src_hbm.at[pl.ds(off, PREP_SUB)], p_src)
        pltpu.sync_copy(prod_hbm.at[pl.ds(off, PREP_SUB)], p_prod)
        pltpu.sync_copy(amt_hbm.at[pl.ds(off, PREP_SUB)], p_amt)

        def vb(j, _):
            sl = pl.ds(j * LANES, LANES)
            dst = pl.ds(k * PREP_SUB + j * LANES, LANES)
            flat = p_src[sl] * NUM_PRODS + (p_prod[sl] - NUM_FIRMS)
            raw_flat[dst] = flat
            raw_amt[dst] = jnp.maximum(p_amt[sl], 0.0)
            ridv = jnp.right_shift(jnp.maximum(flat, 0), 16)
            plsc.addupdate_scatter(cnt, [ridv], ones_i)
            return 0
        lax.fori_loop(0, PREP_SUB // LANES, vb, 0)
        return 0
    lax.fori_loop(0, N_PREP_SUB, prep_sub, 0)

    def scan_body(i, carry):
        sl = pl.ds(i * LANES, LANES)
        v8 = (cnt[sl] + 7) & (-8)
        cs = plsc.cumsum(v8)
        offv = cs - v8 + carry
        cnt8[sl] = v8
        off_start[sl] = offv
        off_run[sl] = offv
        return carry + jnp.sum(v8)
    lax.fori_loop(0, NBKT // LANES, scan_body, 0)

    def fill_body(i, _):
        for u in range(4):
            sorted_flat[pl.ds((i * 4 + u) * LANES, LANES)] = neg1
        return 0
    lax.fori_loop(0, SORT_CAP // (4 * LANES), fill_body, 0)

    def place_body(j, _):
        sl = pl.ds(j * LANES, LANES)
        flat = raw_flat[sl]
        amtv = raw_amt[sl]
        ridv = jnp.right_shift(jnp.maximum(flat, 0), 16)
        rank, _ = plsc.scan_count(ridv)
        basev = plsc.load_gather(off_run, [ridv])
        pos = basev + rank - 1
        plsc.store_scatter(sorted_flat, [pos], flat)
        plsc.store_scatter(sorted_amt, [pos], amtv)
        plsc.addupdate_scatter(off_run, [ridv], ones_i)
        return 0
    lax.fori_loop(0, PREP_PER_TILE // LANES, place_body, 0)

    reg_base = (c * NS + s) * SORT_CAP
    pltpu.sync_copy(sorted_flat, flat_h.at[pl.ds(reg_base, SORT_CAP)])
    pltpu.sync_copy(sorted_amt, amtc_h.at[pl.ds(reg_base, SORT_CAP)])
    pltpu.sync_copy(off_start, tbl_sh.at[pl.ds(s * 2 * NBKT, NBKT)])
    pltpu.sync_copy(cnt8, tbl_sh.at[pl.ds(s * 2 * NBKT + NBKT, NBKT)])
    plsc.subcore_barrier()
    pltpu.sync_copy(tbl_sh, raw_flat.at[pl.ds(0, NS * 2 * NBKT)])

    def pass_body(p, _):
        rid = p * NW + wid

        @pl.when(rid < N_RANGES)
        def _active():
            base = rid * RANGE_WORDS

            def zb(i, _):
                for u in range(4):
                    acc[pl.ds((i * 4 + u) * LANES, LANES)] = zeros_f
                return 0
            lax.fori_loop(0, RANGE_WORDS // (4 * LANES), zb, 0)

            for s2 in range(NS):
                st = pl.multiple_of(plsc.load_gather(
                    raw_flat, [jnp.full((LANES,), s2 * 2 * NBKT + rid, jnp.int32)])[0], 8)
                src_off = (c * NS + s2) * SORT_CAP + st
                pltpu.async_copy(flat_h.at[pl.ds(src_off, SEG_CHUNK)],
                                 sorted_flat.at[pl.ds(s2 * SEG_CHUNK, SEG_CHUNK)], sflat)
                pltpu.async_copy(amtc_h.at[pl.ds(src_off, SEG_CHUNK)],
                                 sorted_amt.at[pl.ds(s2 * SEG_CHUNK, SEG_CHUNK)], samt)
            for s2 in range(NS):
                pltpu.make_async_copy(flat_h.at[pl.ds(0, SEG_CHUNK)],
                                      sorted_flat.at[pl.ds(s2 * SEG_CHUNK, SEG_CHUNK)], sflat).wait()
                pltpu.make_async_copy(amtc_h.at[pl.ds(0, SEG_CHUNK)],
                                      sorted_amt.at[pl.ds(s2 * SEG_CHUNK, SEG_CHUNK)], samt).wait()

            for s2 in range(NS):
                ln = plsc.load_gather(
                    raw_flat, [jnp.full((LANES,), s2 * 2 * NBKT + NBKT + rid, jnp.int32)])[0]
                lnc = jnp.minimum(ln, SEG_CHUNK)
                nv = (lnc + LANES - 1) >> 4

                def sv(j, _):
                    sl = pl.ds(s2 * SEG_CHUNK + j * LANES, LANES)
                    flat = sorted_flat[sl]
                    amtv = sorted_amt[sl]
                    ml = (j * LANES + iota16) < lnc
                    local = flat - base
                    mi = local.astype(jnp.uint32) < jnp.uint32(RANGE_WORDS)
                    m = ml & mi
                    idx = local & (RANGE_WORDS - 1)
                    plsc.addupdate_scatter(acc, [idx], amtv, mask=m)
                    return 0
                lax.fori_loop(0, nv, sv, 0)

                @pl.when(ln > SEG_CHUNK)
                def _slow():
                    st2 = pl.multiple_of(plsc.load_gather(
                        raw_flat, [jnp.full((LANES,), s2 * 2 * NBKT + rid, jnp.int32)])[0], 8)

                    def ch(ci, _):
                        coff = (ci + 1) * SEG_CHUNK
                        src2 = (c * NS + s2) * SORT_CAP + st2 + coff
                        pltpu.sync_copy(flat_h.at[pl.ds(src2, SEG_CHUNK)],
                                        sorted_flat.at[pl.ds(s2 * SEG_CHUNK, SEG_CHUNK)])
                        pltpu.sync_copy(amtc_h.at[pl.ds(src2, SEG_CHUNK)],
                                        sorted_amt.at[pl.ds(s2 * SEG_CHUNK, SEG_CHUNK)])

                        def sv2(j, _):
                            sl = pl.ds(s2 * SEG_CHUNK + j * LANES, LANES)
                            flat = sorted_flat[sl]
                            amtv = sorted_amt[sl]
                            ml = (coff + j * LANES + iota16) < ln
                            local = flat - base
                            mi = local.astype(jnp.uint32) < jnp.uint32(RANGE_WORDS)
                            m = ml & mi
                            idx = local & (RANGE_WORDS - 1)
                            plsc.addupdate_scatter(acc, [idx], amtv, mask=m)
                            return 0
                        lax.fori_loop(0, SEG_CHUNK // LANES, sv2, 0)
                        return 0
                    nch = (ln - 1) >> 9
                    lax.fori_loop(0, nch, ch, 0)

            pltpu.sync_copy(acc, tot_hbm.at[pl.ds(base, RANGE_WORDS)])
        return 0
    lax.fori_loop(0, N_PASSES, pass_body, 0)


@jax.jit
def _sc_scatter(src, prod, amt):
    mesh = plsc.VectorSubcoreMesh(core_axis_name="c", subcore_axis_name="s")
    return pl.kernel(
        _scatter_kernel,
        out_type=(
            jax.ShapeDtypeStruct((TOT_ROWS * NUM_PRODS,), jnp.float32),
            jax.ShapeDtypeStruct((REG_TOTAL,), jnp.int32),
            jax.ShapeDtypeStruct((REG_TOTAL,), jnp.float32),
        ),
        mesh=mesh,
        compiler_params=pltpu.CompilerParams(needs_layout_passes=False),
        scratch_types=[
            pltpu.VMEM((RANGE_WORDS,), jnp.float32),
            pltpu.VMEM((PREP_PER_TILE,), jnp.int32),
            pltpu.VMEM((PREP_PER_TILE,), jnp.float32),
            pltpu.VMEM((SORT_CAP,), jnp.int32),
            pltpu.VMEM((SORT_CAP,), jnp.float32),
            pltpu.VMEM((PREP_SUB,), jnp.int32),
            pltpu.VMEM((PREP_SUB,), jnp.int32),
            pltpu.VMEM((PREP_SUB,), jnp.float32),
            pltpu.VMEM((NBKT,), jnp.int32),
            pltpu.VMEM((NBKT,), jnp.int32),
            pltpu.VMEM((NBKT,), jnp.int32),
            pltpu.VMEM((NBKT,), jnp.int32),
            pltpu.VMEM_SHARED((NS * 2 * NBKT,), jnp.int32),
            pltpu.SemaphoreType.DMA,
            pltpu.SemaphoreType.DMA,
        ],
    )(src, prod, amt)


ROWS_PER_BLOCK = 512
N_BLOCKS = TOT_ROWS // ROWS_PER_BLOCK


def _reduce_kernel(tot_ref, aw_ref, debt_ref, cons_ref):
    i = pl.program_id(0)
    aw = aw_ref[...]
    r = lax.broadcasted_iota(jnp.int32, (NUM_PRODS, NUM_PRODS), 0)
    c = lax.broadcasted_iota(jnp.int32, (NUM_PRODS, NUM_PRODS), 1)
    att = jnp.maximum(jnp.where(r == c, 0.0, aw), 0.0)
    cons = jnp.dot(tot_ref[...], att, preferred_element_type=jnp.float32)
    d = jnp.sum(jnp.maximum(cons - 1.0, 0.0))
    sm = jnp.sum(cons)

    @pl.when(i == 0)
    def _init():
        debt_ref[0, 0] = d
        cons_ref[0, 0] = sm

    @pl.when(i != 0)
    def _acc():
        debt_ref[0, 0] += d
        cons_ref[0, 0] += sm


@jax.jit
def _tc_reduce(totals2d, att_weights):
    return pl.pallas_call(
        _reduce_kernel,
        grid=(N_BLOCKS,),
        in_specs=[
            pl.BlockSpec((ROWS_PER_BLOCK, NUM_PRODS), lambda i: (i, 0)),
            pl.BlockSpec((NUM_PRODS, NUM_PRODS), lambda i: (0, 0)),
        ],
        out_specs=[
            pl.BlockSpec(memory_space=pltpu.SMEM),
            pl.BlockSpec(memory_space=pltpu.SMEM),
        ],
        out_shape=[
            jax.ShapeDtypeStruct((1, 1), jnp.float32),
            jax.ShapeDtypeStruct((1, 1), jnp.float32),
        ],
    )(totals2d, att_weights)


def kernel(src, dst, prod, t, amt, att_weights):
    pad = E_PAD - src.shape[0]
    srcp = jnp.pad(src, (0, pad))
    prodp = jnp.pad(prod, (0, pad))
    amtp = jnp.pad(amt, (0, pad))
    totals, _fh, _ah = _sc_scatter(srcp, prodp, amtp)
    totals2d = totals.reshape(TOT_ROWS, NUM_PRODS)
    debt_sum, cons_sum = _tc_reduce(totals2d, att_weights)
    n = src.shape[0]
    debt_loss = DEBT_PENALTY * debt_sum[0, 0] / NUM_FIRMS
    consump_rwd = CONSUMPTION_REWARD * cons_sum[0, 0] / NUM_FIRMS
    inv_loss = debt_loss - consump_rwd
    return (inv_loss / n, debt_loss / n, consump_rwd / n)


# speedup vs baseline: 5.8190x; 1.5678x over previous
import jax
import jax.numpy as jnp
from jax import lax
from jax.experimental import pallas as pl
from jax.experimental.pallas import tpu as pltpu
from jax.experimental.pallas import tpu_sc as plsc

NUM_FIRMS = 50000
NUM_PRODS = 256
E = 200000
DEBT_PENALTY = 5.0
CONSUMPTION_REWARD = 4.0

NC = 2
NS = 16
NW = NC * NS
LANES = 16

RANGE_WORDS = 65536
N_RANGES = 196
N_PASSES = 7
TOT_ROWS = (N_RANGES * RANGE_WORDS) // NUM_PRODS

E_PAD = 200704
PREP_PER_TILE = E_PAD // NS
PREP_SUB = 1568
N_PREP_SUB = PREP_PER_TILE // PREP_SUB

SORT_CAP = 14336
SEG_CHUNK = 512
REG_TOTAL = NC * NS * SORT_CAP + 2 * SEG_CHUNK
NBKT = 256


def _scatter_kernel(src_hbm, prod_hbm, amt_hbm,
                    tot_hbm, flat_h, amtc_h,
                    acc, raw_flat, raw_amt, sorted_flat, sorted_amt,
                    p_src, p_prod, p_amt,
                    cnt, cnt8, off_start, off_run, tbl_sh,
                    sflat, samt):
    s = lax.axis_index("s")
    c = lax.axis_index("c")
    wid = s * NC + c
    ones_i = jnp.ones((LANES,), jnp.int32)
    zeros_i = jnp.zeros((LANES,), jnp.int32)
    zeros_f = jnp.zeros((LANES,), jnp.float32)
    neg1 = jnp.full((LANES,), -1, jnp.int32)
    iota16 = lax.iota(jnp.int32, LANES)

    for i in range(NBKT // LANES):
        cnt[pl.ds(i * LANES, LANES)] = zeros_i

    def prep_sub(k, _):
        off = s * PREP_PER_TILE + k * PREP_SUB
        pltpu.sync_copy(src_hbm.at[pl.ds(off, PREP_SUB)], p_src)
        pltpu.sync_copy(prod_hbm.at[pl.ds(off, PREP_SUB)], p_prod)
        pltpu.sync_copy(amt_hbm.at[pl.ds(off, PREP_SUB)], p_amt)

        def vb(j, _):
            sl = pl.ds(j * LANES, LANES)
            dst = pl.ds(k * PREP_SUB + j * LANES, LANES)
            flat = p_src[sl] * NUM_PRODS + (p_prod[sl] - NUM_FIRMS)
            raw_flat[dst] = flat
            raw_amt[dst] = jnp.maximum(p_amt[sl], 0.0)
            ridv = jnp.right_shift(jnp.maximum(flat, 0), 16)
            plsc.addupdate_scatter(cnt, [ridv], ones_i)
            return 0
        lax.fori_loop(0, PREP_SUB // LANES, vb, 0)
        return 0
    lax.fori_loop(0, N_PREP_SUB, prep_sub, 0)

    def scan_body(i, carry):
        sl = pl.ds(i * LANES, LANES)
        v8 = (cnt[sl] + 7) & (-8)
        cs = plsc.cumsum(v8)
        offv = cs - v8 + carry
        cnt8[sl] = v8
        off_start[sl] = offv
        off_run[sl] = offv
        return carry + jnp.sum(v8)
    lax.fori_loop(0, NBKT // LANES, scan_body, 0)

    def fill_body(i, _):
        for u in range(4):
            sorted_flat[pl.ds((i * 4 + u) * LANES, LANES)] = neg1
        return 0
    lax.fori_loop(0, SORT_CAP // (4 * LANES), fill_body, 0)

    def place_body(j, _):
        sl = pl.ds(j * LANES, LANES)
        flat = raw_flat[sl]
        amtv = raw_amt[sl]
        ridv = jnp.right_shift(jnp.maximum(flat, 0), 16)
        rank, _ = plsc.scan_count(ridv)
        basev = plsc.load_gather(off_run, [ridv])
        pos = basev + rank - 1
        plsc.store_scatter(sorted_flat, [pos], flat)
        plsc.store_scatter(sorted_amt, [pos], amtv)
        plsc.addupdate_scatter(off_run, [ridv], ones_i)
        return 0
    lax.fori_loop(0, PREP_PER_TILE // LANES, place_body, 0)

    reg_base = (c * NS + s) * SORT_CAP
    pltpu.sync_copy(sorted_flat, flat_h.at[pl.ds(reg_base, SORT_CAP)])
    pltpu.sync_copy(sorted_amt, amtc_h.at[pl.ds(reg_base, SORT_CAP)])
    pltpu.sync_copy(off_start, tbl_sh.at[pl.ds(s * 2 * NBKT, NBKT)])
    pltpu.sync_copy(cnt8, tbl_sh.at[pl.ds(s * 2 * NBKT + NBKT, NBKT)])
    plsc.subcore_barrier()
    pltpu.sync_copy(tbl_sh, raw_flat.at[pl.ds(0, NS * 2 * NBKT)])

    def pass_body(p, _):
        rid = p * NW + wid

        @pl.when(rid < N_RANGES)
        def _active():
            base = rid * RANGE_WORDS

            def zb(i, _):
                for u in range(NUM_PRODS // LANES):
                    acc[i, pl.ds(u * LANES, LANES)] = zeros_f
                return 0
            lax.fori_loop(0, RANGE_WORDS // NUM_PRODS, zb, 0)

            for s2 in range(NS):
                st = pl.multiple_of(plsc.load_gather(
                    raw_flat, [jnp.full((LANES,), s2 * 2 * NBKT + rid, jnp.int32)])[0], 8)
                src_off = (c * NS + s2) * SORT_CAP + st
                pltpu.async_copy(flat_h.at[pl.ds(src_off, SEG_CHUNK)],
                                 sorted_flat.at[pl.ds(s2 * SEG_CHUNK, SEG_CHUNK)], sflat)
                pltpu.async_copy(amtc_h.at[pl.ds(src_off, SEG_CHUNK)],
                                 sorted_amt.at[pl.ds(s2 * SEG_CHUNK, SEG_CHUNK)], samt)
            for s2 in range(NS):
                pltpu.make_async_copy(flat_h.at[pl.ds(0, SEG_CHUNK)],
                                      sorted_flat.at[pl.ds(s2 * SEG_CHUNK, SEG_CHUNK)], sflat).wait()
                pltpu.make_async_copy(amtc_h.at[pl.ds(0, SEG_CHUNK)],
                                      sorted_amt.at[pl.ds(s2 * SEG_CHUNK, SEG_CHUNK)], samt).wait()

            for s2 in range(NS):
                ln = plsc.load_gather(
                    raw_flat, [jnp.full((LANES,), s2 * 2 * NBKT + NBKT + rid, jnp.int32)])[0]
                lnc = jnp.minimum(ln, SEG_CHUNK)
                nv = (lnc + LANES - 1) >> 4

                def sv(j, _):
                    sl = pl.ds(s2 * SEG_CHUNK + j * LANES, LANES)
                    flat = sorted_flat[sl]
                    amtv = sorted_amt[sl]
                    ml = (j * LANES + iota16) < lnc
                    local = flat - base
                    mi = local.astype(jnp.uint32) < jnp.uint32(RANGE_WORDS)
                    m = ml & mi
                    lidx = local & (RANGE_WORDS - 1)
                    rowv = jnp.right_shift(lidx, 8)
                    colv = lidx & (NUM_PRODS - 1)
                    plsc.addupdate_scatter(acc, [rowv, colv], amtv, mask=m)
                    return 0
                lax.fori_loop(0, nv, sv, 0)

                @pl.when(ln > SEG_CHUNK)
                def _slow():
                    st2 = pl.multiple_of(plsc.load_gather(
                        raw_flat, [jnp.full((LANES,), s2 * 2 * NBKT + rid, jnp.int32)])[0], 8)

                    def ch(ci, _):
                        coff = (ci + 1) * SEG_CHUNK
                        src2 = (c * NS + s2) * SORT_CAP + st2 + coff
                        pltpu.sync_copy(flat_h.at[pl.ds(src2, SEG_CHUNK)],
                                        sorted_flat.at[pl.ds(s2 * SEG_CHUNK, SEG_CHUNK)])
                        pltpu.sync_copy(amtc_h.at[pl.ds(src2, SEG_CHUNK)],
                                        sorted_amt.at[pl.ds(s2 * SEG_CHUNK, SEG_CHUNK)])

                        def sv2(j, _):
                            sl = pl.ds(s2 * SEG_CHUNK + j * LANES, LANES)
                            flat = sorted_flat[sl]
                            amtv = sorted_amt[sl]
                            ml = (coff + j * LANES + iota16) < ln
                            local = flat - base
                            mi = local.astype(jnp.uint32) < jnp.uint32(RANGE_WORDS)
                            m = ml & mi
                            lidx = local & (RANGE_WORDS - 1)
                            rowv = jnp.right_shift(lidx, 8)
                            colv = lidx & (NUM_PRODS - 1)
                            plsc.addupdate_scatter(acc, [rowv, colv], amtv, mask=m)
                            return 0
                        lax.fori_loop(0, SEG_CHUNK // LANES, sv2, 0)
                        return 0
                    nch = (ln - 1) >> 9
                    lax.fori_loop(0, nch, ch, 0)

            pltpu.sync_copy(
                acc, tot_hbm.at[pl.ds(rid * (RANGE_WORDS // NUM_PRODS),
                                      RANGE_WORDS // NUM_PRODS)])
        return 0
    lax.fori_loop(0, N_PASSES, pass_body, 0)


@jax.jit
def _sc_scatter(src, prod, amt):
    mesh = plsc.VectorSubcoreMesh(core_axis_name="c", subcore_axis_name="s")
    return pl.kernel(
        _scatter_kernel,
        out_type=(
            jax.ShapeDtypeStruct((TOT_ROWS, NUM_PRODS), jnp.float32),
            jax.ShapeDtypeStruct((REG_TOTAL,), jnp.int32),
            jax.ShapeDtypeStruct((REG_TOTAL,), jnp.float32),
        ),
        mesh=mesh,
        compiler_params=pltpu.CompilerParams(needs_layout_passes=False),
        scratch_types=[
            pltpu.VMEM((RANGE_WORDS // NUM_PRODS, NUM_PRODS), jnp.float32),
            pltpu.VMEM((PREP_PER_TILE,), jnp.int32),
            pltpu.VMEM((PREP_PER_TILE,), jnp.float32),
            pltpu.VMEM((SORT_CAP,), jnp.int32),
            pltpu.VMEM((SORT_CAP,), jnp.float32),
            pltpu.VMEM((PREP_SUB,), jnp.int32),
            pltpu.VMEM((PREP_SUB,), jnp.int32),
            pltpu.VMEM((PREP_SUB,), jnp.float32),
            pltpu.VMEM((NBKT,), jnp.int32),
            pltpu.VMEM((NBKT,), jnp.int32),
            pltpu.VMEM((NBKT,), jnp.int32),
            pltpu.VMEM((NBKT,), jnp.int32),
            pltpu.VMEM_SHARED((NS * 2 * NBKT,), jnp.int32),
            pltpu.SemaphoreType.DMA,
            pltpu.SemaphoreType.DMA,
        ],
    )(src, prod, amt)


ROWS_PER_BLOCK = 1024
N_BLOCKS = TOT_ROWS // ROWS_PER_BLOCK


def _reduce_kernel(tot_ref, aw_ref, debt_ref, cons_ref):
    i = pl.program_id(0)
    aw = aw_ref[...]
    r = lax.broadcasted_iota(jnp.int32, (NUM_PRODS, NUM_PRODS), 0)
    c = lax.broadcasted_iota(jnp.int32, (NUM_PRODS, NUM_PRODS), 1)
    att = jnp.maximum(jnp.where(r == c, 0.0, aw), 0.0)
    cons = jnp.dot(tot_ref[...], att, preferred_element_type=jnp.float32)
    d = jnp.sum(jnp.maximum(cons - 1.0, 0.0))
    sm = jnp.sum(cons)

    @pl.when(i == 0)
    def _init():
        debt_ref[0, 0] = d
        cons_ref[0, 0] = sm

    @pl.when(i != 0)
    def _acc():
        debt_ref[0, 0] += d
        cons_ref[0, 0] += sm


@jax.jit
def _tc_reduce(totals2d, att_weights):
    return pl.pallas_call(
        _reduce_kernel,
        grid=(N_BLOCKS,),
        in_specs=[
            pl.BlockSpec((ROWS_PER_BLOCK, NUM_PRODS), lambda i: (i, 0)),
            pl.BlockSpec((NUM_PRODS, NUM_PRODS), lambda i: (0, 0)),
        ],
        out_specs=[
            pl.BlockSpec(memory_space=pltpu.SMEM),
            pl.BlockSpec(memory_space=pltpu.SMEM),
        ],
        out_shape=[
            jax.ShapeDtypeStruct((1, 1), jnp.float32),
            jax.ShapeDtypeStruct((1, 1), jnp.float32),
        ],
    )(totals2d, att_weights)


def kernel(src, dst, prod, t, amt, att_weights):
    pad = E_PAD - src.shape[0]
    srcp = jnp.pad(src, (0, pad))
    prodp = jnp.pad(prod, (0, pad))
    amtp = jnp.pad(amt, (0, pad))
    totals2d, _fh, _ah = _sc_scatter(srcp, prodp, amtp)
    debt_sum, cons_sum = _tc_reduce(totals2d, att_weights)
    n = src.shape[0]
    debt_loss = DEBT_PENALTY * debt_sum[0, 0] / NUM_FIRMS
    consump_rwd = CONSUMPTION_REWARD * cons_sum[0, 0] / NUM_FIRMS
    inv_loss = debt_loss - consump_rwd
    return (inv_loss / n, debt_loss / n, consump_rwd / n)


# speedup vs baseline: 6.6196x; 1.1376x over previous
import jax
import jax.numpy as jnp
from jax import lax
from jax.experimental import pallas as pl
from jax.experimental.pallas import tpu as pltpu
from jax.experimental.pallas import tpu_sc as plsc

NUM_FIRMS = 50000
NUM_PRODS = 256
E = 200000
DEBT_PENALTY = 5.0
CONSUMPTION_REWARD = 4.0

NC = 2
NS = 16
NW = NC * NS
LANES = 16

RANGE_WORDS = 65536
N_RANGES = 196
N_PASSES = 7
TOT_ROWS = (N_RANGES * RANGE_WORDS) // NUM_PRODS

E_PAD = 200704
PREP_PER_TILE = E_PAD // NS
PREP_SUB = 1568
N_PREP_SUB = PREP_PER_TILE // PREP_SUB

SORT_CAP = 14336
SEG_CHUNK = 512
REG_TOTAL = NC * NS * SORT_CAP + 2 * SEG_CHUNK
NBKT = 256


def _scatter_kernel(src_hbm, prod_hbm, amt_hbm,
                    tot_hbm, flat_h, amtc_h,
                    acc, raw_flat, raw_amt, sorted_flat, sorted_amt,
                    p_src0, p_prod0, p_amt0, p_src1, p_prod1, p_amt1,
                    cnt, cnt8, off_start, off_run, tbl_sh,
                    sflat, samt, sp0, sp1, swb):
    s = lax.axis_index("s")
    c = lax.axis_index("c")
    wid = s * NC + c
    ones_i = jnp.ones((LANES,), jnp.int32)
    zeros_i = jnp.zeros((LANES,), jnp.int32)
    zeros_f = jnp.zeros((LANES,), jnp.float32)
    neg1 = jnp.full((LANES,), -1, jnp.int32)
    iota16 = lax.iota(jnp.int32, LANES)

    for i in range(NBKT // LANES):
        cnt[pl.ds(i * LANES, LANES)] = zeros_i

    tile_off = s * PREP_PER_TILE

    def issue_prep(k, bs, bp, ba, sem):
        off = tile_off + k * PREP_SUB
        pltpu.async_copy(src_hbm.at[pl.ds(off, PREP_SUB)], bs, sem)
        pltpu.async_copy(prod_hbm.at[pl.ds(off, PREP_SUB)], bp, sem)
        pltpu.async_copy(amt_hbm.at[pl.ds(off, PREP_SUB)], ba, sem)

    def wait_prep(bs, bp, ba, sem):
        pltpu.make_async_copy(src_hbm.at[pl.ds(0, PREP_SUB)], bs, sem).wait()
        pltpu.make_async_copy(prod_hbm.at[pl.ds(0, PREP_SUB)], bp, sem).wait()
        pltpu.make_async_copy(amt_hbm.at[pl.ds(0, PREP_SUB)], ba, sem).wait()

    def compute_prep(k, bs, bp, ba):
        def vb(j, _):
            sl = pl.ds(j * LANES, LANES)
            dst = pl.ds(k * PREP_SUB + j * LANES, LANES)
            flat = bs[sl] * NUM_PRODS + (bp[sl] - NUM_FIRMS)
            raw_flat[dst] = flat
            raw_amt[dst] = jnp.maximum(ba[sl], 0.0)
            ridv = jnp.right_shift(jnp.maximum(flat, 0), 16)
            plsc.addupdate_scatter(cnt, [ridv], ones_i)
            return 0
        lax.fori_loop(0, PREP_SUB // LANES, vb, 0)

    issue_prep(0, p_src0, p_prod0, p_amt0, sp0)
    issue_prep(1, p_src1, p_prod1, p_amt1, sp1)

    def prep_pair(k2, _):
        k0 = k2 * 2
        wait_prep(p_src0, p_prod0, p_amt0, sp0)
        compute_prep(k0, p_src0, p_prod0, p_amt0)

        @pl.when(k2 < N_PREP_SUB // 2 - 1)
        def _i0():
            issue_prep(k0 + 2, p_src0, p_prod0, p_amt0, sp0)
        wait_prep(p_src1, p_prod1, p_amt1, sp1)
        compute_prep(k0 + 1, p_src1, p_prod1, p_amt1)

        @pl.when(k2 < N_PREP_SUB // 2 - 1)
        def _i1():
            issue_prep(k0 + 3, p_src1, p_prod1, p_amt1, sp1)
        return 0
    lax.fori_loop(0, N_PREP_SUB // 2, prep_pair, 0)

    def scan_body(i, carry):
        sl = pl.ds(i * LANES, LANES)
        v8 = (cnt[sl] + 7) & (-8)
        cs = plsc.cumsum(v8)
        offv = cs - v8 + carry
        cnt8[sl] = v8
        off_start[sl] = offv
        off_run[sl] = offv
        return carry + jnp.sum(v8)
    lax.fori_loop(0, NBKT // LANES, scan_body, 0)

    def fill_body(i, _):
        for u in range(4):
            sorted_flat[pl.ds((i * 4 + u) * LANES, LANES)] = neg1
        return 0
    lax.fori_loop(0, SORT_CAP // (4 * LANES), fill_body, 0)

    def place_body(j, _):
        sl = pl.ds(j * LANES, LANES)
        flat = raw_flat[sl]
        amtv = raw_amt[sl]
        ridv = jnp.right_shift(jnp.maximum(flat, 0), 16)
        rank, _ = plsc.scan_count(ridv)
        basev = plsc.load_gather(off_run, [ridv])
        pos = basev + rank - 1
        plsc.store_scatter(sorted_flat, [pos], flat)
        plsc.store_scatter(sorted_amt, [pos], amtv)
        plsc.addupdate_scatter(off_run, [ridv], ones_i)
        return 0
    lax.fori_loop(0, PREP_PER_TILE // LANES, place_body, 0)

    reg_base = (c * NS + s) * SORT_CAP
    pltpu.sync_copy(sorted_flat, flat_h.at[pl.ds(reg_base, SORT_CAP)])
    pltpu.sync_copy(sorted_amt, amtc_h.at[pl.ds(reg_base, SORT_CAP)])
    pltpu.sync_copy(off_start, tbl_sh.at[pl.ds(s * 2 * NBKT, NBKT)])
    pltpu.sync_copy(cnt8, tbl_sh.at[pl.ds(s * 2 * NBKT + NBKT, NBKT)])
    plsc.subcore_barrier()
    pltpu.sync_copy(tbl_sh, raw_flat.at[pl.ds(0, NS * 2 * NBKT)])

    def pass_body(p, _):
        rid = p * NW + wid

        @pl.when(rid < N_RANGES)
        def _active():
            base = rid * RANGE_WORDS

            for s2 in range(NS):
                st = pl.multiple_of(plsc.load_gather(
                    raw_flat, [jnp.full((LANES,), s2 * 2 * NBKT + rid, jnp.int32)])[0], 8)
                src_off = (c * NS + s2) * SORT_CAP + st
                pltpu.async_copy(flat_h.at[pl.ds(src_off, SEG_CHUNK)],
                                 sorted_flat.at[pl.ds(s2 * SEG_CHUNK, SEG_CHUNK)], sflat)
                pltpu.async_copy(amtc_h.at[pl.ds(src_off, SEG_CHUNK)],
                                 sorted_amt.at[pl.ds(s2 * SEG_CHUNK, SEG_CHUNK)], samt)
            @pl.when(p > 0)
            def _drain_wb():
                pltpu.make_async_copy(
                    acc, tot_hbm.at[pl.ds(0, RANGE_WORDS // NUM_PRODS)], swb).wait()

            def zb(i, _):
                for u in range(NUM_PRODS // LANES):
                    acc[i, pl.ds(u * LANES, LANES)] = zeros_f
                return 0
            lax.fori_loop(0, RANGE_WORDS // NUM_PRODS, zb, 0)

            for s2 in range(NS):
                pltpu.make_async_copy(flat_h.at[pl.ds(0, SEG_CHUNK)],
                                      sorted_flat.at[pl.ds(s2 * SEG_CHUNK, SEG_CHUNK)], sflat).wait()
                pltpu.make_async_copy(amtc_h.at[pl.ds(0, SEG_CHUNK)],
                                      sorted_amt.at[pl.ds(s2 * SEG_CHUNK, SEG_CHUNK)], samt).wait()

            for s2 in range(NS):
                ln = plsc.load_gather(
                    raw_flat, [jnp.full((LANES,), s2 * 2 * NBKT + NBKT + rid, jnp.int32)])[0]
                lnc = jnp.minimum(ln, SEG_CHUNK)
                nv = (lnc + LANES - 1) >> 4

                def sv(j, _):
                    sl = pl.ds(s2 * SEG_CHUNK + j * LANES, LANES)
                    flat = sorted_flat[sl]
                    amtv = sorted_amt[sl]
                    ml = (j * LANES + iota16) < lnc
                    local = flat - base
                    mi = local.astype(jnp.uint32) < jnp.uint32(RANGE_WORDS)
                    m = ml & mi
                    lidx = local & (RANGE_WORDS - 1)
                    rowv = jnp.right_shift(lidx, 8)
                    colv = lidx & (NUM_PRODS - 1)
                    plsc.addupdate_scatter(acc, [rowv, colv], amtv, mask=m)
                    return 0
                lax.fori_loop(0, nv, sv, 0)

                @pl.when(ln > SEG_CHUNK)
                def _slow():
                    st2 = pl.multiple_of(plsc.load_gather(
                        raw_flat, [jnp.full((LANES,), s2 * 2 * NBKT + rid, jnp.int32)])[0], 8)

                    def ch(ci, _):
                        coff = (ci + 1) * SEG_CHUNK
                        src2 = (c * NS + s2) * SORT_CAP + st2 + coff
                        pltpu.sync_copy(flat_h.at[pl.ds(src2, SEG_CHUNK)],
                                        sorted_flat.at[pl.ds(s2 * SEG_CHUNK, SEG_CHUNK)])
                        pltpu.sync_copy(amtc_h.at[pl.ds(src2, SEG_CHUNK)],
                                        sorted_amt.at[pl.ds(s2 * SEG_CHUNK, SEG_CHUNK)])

                        def sv2(j, _):
                            sl = pl.ds(s2 * SEG_CHUNK + j * LANES, LANES)
                            flat = sorted_flat[sl]
                            amtv = sorted_amt[sl]
                            ml = (coff + j * LANES + iota16) < ln
                            local = flat - base
                            mi = local.astype(jnp.uint32) < jnp.uint32(RANGE_WORDS)
                            m = ml & mi
                            lidx = local & (RANGE_WORDS - 1)
                            rowv = jnp.right_shift(lidx, 8)
                            colv = lidx & (NUM_PRODS - 1)
                            plsc.addupdate_scatter(acc, [rowv, colv], amtv, mask=m)
                            return 0
                        lax.fori_loop(0, SEG_CHUNK // LANES, sv2, 0)
                        return 0
                    nch = (ln - 1) >> 9
                    lax.fori_loop(0, nch, ch, 0)

            pltpu.async_copy(
                acc, tot_hbm.at[pl.ds(rid * (RANGE_WORDS // NUM_PRODS),
                                      RANGE_WORDS // NUM_PRODS)], swb)
        return 0
    lax.fori_loop(0, N_PASSES, pass_body, 0)
    pltpu.make_async_copy(
        acc, tot_hbm.at[pl.ds(0, RANGE_WORDS // NUM_PRODS)], swb).wait()


@jax.jit
def _sc_scatter(src, prod, amt):
    mesh = plsc.VectorSubcoreMesh(core_axis_name="c", subcore_axis_name="s")
    return pl.kernel(
        _scatter_kernel,
        out_type=(
            jax.ShapeDtypeStruct((TOT_ROWS, NUM_PRODS), jnp.float32),
            jax.ShapeDtypeStruct((REG_TOTAL,), jnp.int32),
            jax.ShapeDtypeStruct((REG_TOTAL,), jnp.float32),
        ),
        mesh=mesh,
        compiler_params=pltpu.CompilerParams(needs_layout_passes=False),
        scratch_types=[
            pltpu.VMEM((RANGE_WORDS // NUM_PRODS, NUM_PRODS), jnp.float32),
            pltpu.VMEM((PREP_PER_TILE,), jnp.int32),
            pltpu.VMEM((PREP_PER_TILE,), jnp.float32),
            pltpu.VMEM((SORT_CAP,), jnp.int32),
            pltpu.VMEM((SORT_CAP,), jnp.float32),
            pltpu.VMEM((PREP_SUB,), jnp.int32),
            pltpu.VMEM((PREP_SUB,), jnp.int32),
            pltpu.VMEM((PREP_SUB,), jnp.float32),
            pltpu.VMEM((PREP_SUB,), jnp.int32),
            pltpu.VMEM((PREP_SUB,), jnp.int32),
            pltpu.VMEM((PREP_SUB,), jnp.float32),
            pltpu.VMEM((NBKT,), jnp.int32),
            pltpu.VMEM((NBKT,), jnp.int32),
            pltpu.VMEM((NBKT,), jnp.int32),
            pltpu.VMEM((NBKT,), jnp.int32),
            pltpu.VMEM_SHARED((NS * 2 * NBKT,), jnp.int32),
            pltpu.SemaphoreType.DMA,
            pltpu.SemaphoreType.DMA,
            pltpu.SemaphoreType.DMA,
            pltpu.SemaphoreType.DMA,
            pltpu.SemaphoreType.DMA,
        ],
    )(src, prod, amt)


ROWS_PER_BLOCK = 1024
N_BLOCKS = TOT_ROWS // ROWS_PER_BLOCK


def _reduce_kernel(tot_ref, aw_ref, debt_ref, cons_ref):
    i = pl.program_id(0)
    aw = aw_ref[...]
    r = lax.broadcasted_iota(jnp.int32, (NUM_PRODS, NUM_PRODS), 0)
    c = lax.broadcasted_iota(jnp.int32, (NUM_PRODS, NUM_PRODS), 1)
    att = jnp.maximum(jnp.where(r == c, 0.0, aw), 0.0)
    cons = jnp.dot(tot_ref[...], att, preferred_element_type=jnp.float32)
    d = jnp.sum(jnp.maximum(cons - 1.0, 0.0))
    sm = jnp.sum(cons)

    @pl.when(i == 0)
    def _init():
        debt_ref[0, 0] = d
        cons_ref[0, 0] = sm

    @pl.when(i != 0)
    def _acc():
        debt_ref[0, 0] += d
        cons_ref[0, 0] += sm


@jax.jit
def _tc_reduce(totals2d, att_weights):
    return pl.pallas_call(
        _reduce_kernel,
        grid=(N_BLOCKS,),
        in_specs=[
            pl.BlockSpec((ROWS_PER_BLOCK, NUM_PRODS), lambda i: (i, 0)),
            pl.BlockSpec((NUM_PRODS, NUM_PRODS), lambda i: (0, 0)),
        ],
        out_specs=[
            pl.BlockSpec(memory_space=pltpu.SMEM),
            pl.BlockSpec(memory_space=pltpu.SMEM),
        ],
        out_shape=[
            jax.ShapeDtypeStruct((1, 1), jnp.float32),
            jax.ShapeDtypeStruct((1, 1), jnp.float32),
        ],
    )(totals2d, att_weights)


def kernel(src, dst, prod, t, amt, att_weights):
    pad = E_PAD - src.shape[0]
    srcp = jnp.pad(src, (0, pad))
    prodp = jnp.pad(prod, (0, pad))
    amtp = jnp.pad(amt, (0, pad))
    totals2d, _fh, _ah = _sc_scatter(srcp, prodp, amtp)
    debt_sum, cons_sum = _tc_reduce(totals2d, att_weights)
    n = src.shape[0]
    debt_loss = DEBT_PENALTY * debt_sum[0, 0] / NUM_FIRMS
    consump_rwd = CONSUMPTION_REWARD * cons_sum[0, 0] / NUM_FIRMS
    inv_loss = debt_loss - consump_rwd
    return (inv_loss / n, debt_loss / n, consump_rwd / n)


# speedup vs baseline: 7.6889x; 1.1615x over previous
import jax
import jax.numpy as jnp
from jax import lax
from jax.experimental import pallas as pl
from jax.experimental.pallas import tpu as pltpu
from jax.experimental.pallas import tpu_sc as plsc

NUM_FIRMS = 50000
NUM_PRODS = 256
E = 200000
DEBT_PENALTY = 5.0
CONSUMPTION_REWARD = 4.0

NC = 2
NS = 16
NW = NC * NS
LANES = 16

RANGE_WORDS = 65536
N_RANGES = 196
N_PASSES = 7
TOT_ROWS = (N_RANGES * RANGE_WORDS) // NUM_PRODS

E_PAD = 200704
PREP_PER_TILE = E_PAD // NS
PREP_SUB = 1568
N_PREP_SUB = PREP_PER_TILE // PREP_SUB

SORT_CAP = 14336
SEG_CHUNK = 512
REG_TOTAL = NC * NS * SORT_CAP + 2 * SEG_CHUNK
NBKT = 256


def _scatter_kernel(src_hbm, prod_hbm, amt_hbm,
                    tot_hbm, flat_h, amtc_h,
                    acc, raw_flat, raw_amt, sorted_flat, sorted_amt,
                    p_src0, p_prod0, p_amt0, p_src1, p_prod1, p_amt1,
                    cnt, cnt8, off_start, off_run, tbl_sh,
                    sflat, samt, sp0, sp1, swb):
    s = lax.axis_index("s")
    c = lax.axis_index("c")
    wid = s * NC + c
    ones_i = jnp.ones((LANES,), jnp.int32)
    zeros_i = jnp.zeros((LANES,), jnp.int32)
    zeros_f = jnp.zeros((LANES,), jnp.float32)
    neg1 = jnp.full((LANES,), -1, jnp.int32)
    iota16 = lax.iota(jnp.int32, LANES)

    for i in range(NBKT // LANES):
        cnt[pl.ds(i * LANES, LANES)] = zeros_i

    tile_off = s * PREP_PER_TILE

    def issue_prep(k, bs, bp, ba, sem):
        off = tile_off + k * PREP_SUB
        pltpu.async_copy(src_hbm.at[pl.ds(off, PREP_SUB)], bs, sem)
        pltpu.async_copy(prod_hbm.at[pl.ds(off, PREP_SUB)], bp, sem)
        pltpu.async_copy(amt_hbm.at[pl.ds(off, PREP_SUB)], ba, sem)

    def wait_prep(bs, bp, ba, sem):
        pltpu.make_async_copy(src_hbm.at[pl.ds(0, PREP_SUB)], bs, sem).wait()
        pltpu.make_async_copy(prod_hbm.at[pl.ds(0, PREP_SUB)], bp, sem).wait()
        pltpu.make_async_copy(amt_hbm.at[pl.ds(0, PREP_SUB)], ba, sem).wait()

    def compute_prep(k, bs, bp, ba):
        def vb(j, _):
            sl = pl.ds(j * LANES, LANES)
            dst = pl.ds(k * PREP_SUB + j * LANES, LANES)
            flat = bs[sl] * NUM_PRODS + (bp[sl] - NUM_FIRMS)
            raw_flat[dst] = flat
            raw_amt[dst] = jnp.maximum(ba[sl], 0.0)
            ridv = jnp.right_shift(jnp.maximum(flat, 0), 16)
            plsc.addupdate_scatter(cnt, [ridv], ones_i)
            return 0
        lax.fori_loop(0, PREP_SUB // LANES, vb, 0)

    issue_prep(0, p_src0, p_prod0, p_amt0, sp0)
    issue_prep(1, p_src1, p_prod1, p_amt1, sp1)

    def prep_pair(k2, _):
        k0 = k2 * 2
        wait_prep(p_src0, p_prod0, p_amt0, sp0)
        compute_prep(k0, p_src0, p_prod0, p_amt0)

        @pl.when(k2 < N_PREP_SUB // 2 - 1)
        def _i0():
            issue_prep(k0 + 2, p_src0, p_prod0, p_amt0, sp0)
        wait_prep(p_src1, p_prod1, p_amt1, sp1)
        compute_prep(k0 + 1, p_src1, p_prod1, p_amt1)

        @pl.when(k2 < N_PREP_SUB // 2 - 1)
        def _i1():
            issue_prep(k0 + 3, p_src1, p_prod1, p_amt1, sp1)
        return 0
    lax.fori_loop(0, N_PREP_SUB // 2, prep_pair, 0)

    def scan_body(i, carry):
        sl = pl.ds(i * LANES, LANES)
        v8 = (cnt[sl] + 7) & (-8)
        cs = plsc.cumsum(v8)
        offv = cs - v8 + carry
        cnt8[sl] = v8
        off_start[sl] = offv
        off_run[sl] = offv
        return carry + jnp.sum(v8)
    lax.fori_loop(0, NBKT // LANES, scan_body, 0)

    def fill_body(i, _):
        for u in range(4):
            sorted_flat[pl.ds((i * 4 + u) * LANES, LANES)] = neg1
        return 0
    lax.fori_loop(0, SORT_CAP // (4 * LANES), fill_body, 0)

    def place_body(j, _):
        sl = pl.ds(j * LANES, LANES)
        flat = raw_flat[sl]
        amtv = raw_amt[sl]
        ridv = jnp.right_shift(jnp.maximum(flat, 0), 16)
        rank, _ = plsc.scan_count(ridv)
        basev = plsc.load_gather(off_run, [ridv])
        pos = basev + rank - 1
        plsc.store_scatter(sorted_flat, [pos], flat)
        plsc.store_scatter(sorted_amt, [pos], amtv)
        plsc.addupdate_scatter(off_run, [ridv], ones_i)
        return 0
    lax.fori_loop(0, PREP_PER_TILE // LANES, place_body, 0)

    reg_base = (c * NS + s) * SORT_CAP
    pltpu.sync_copy(sorted_flat, flat_h.at[pl.ds(reg_base, SORT_CAP)])
    pltpu.sync_copy(sorted_amt, amtc_h.at[pl.ds(reg_base, SORT_CAP)])
    pltpu.sync_copy(off_start, tbl_sh.at[pl.ds(s * 2 * NBKT, NBKT)])
    pltpu.sync_copy(cnt8, tbl_sh.at[pl.ds(s * 2 * NBKT + NBKT, NBKT)])
    plsc.subcore_barrier()
    pltpu.sync_copy(tbl_sh, raw_flat.at[pl.ds(0, NS * 2 * NBKT)])

    def pass_body(p, _):
        rid = p * NW + wid

        @pl.when(rid < N_RANGES)
        def _active():
            base = rid * RANGE_WORDS

            for s2 in range(NS):
                st = pl.multiple_of(plsc.load_gather(
                    raw_flat, [jnp.full((LANES,), s2 * 2 * NBKT + rid, jnp.int32)])[0], 8)
                src_off = (c * NS + s2) * SORT_CAP + st
                pltpu.async_copy(flat_h.at[pl.ds(src_off, SEG_CHUNK)],
                                 sorted_flat.at[pl.ds(s2 * SEG_CHUNK, SEG_CHUNK)], sflat)
                pltpu.async_copy(amtc_h.at[pl.ds(src_off, SEG_CHUNK)],
                                 sorted_amt.at[pl.ds(s2 * SEG_CHUNK, SEG_CHUNK)], samt)
            @pl.when(p > 0)
            def _drain_wb():
                pltpu.make_async_copy(
                    acc, tot_hbm.at[pl.ds(0, RANGE_WORDS // NUM_PRODS)], swb).wait()

            def zb(i, _):
                for u in range(NUM_PRODS // LANES):
                    acc[i, pl.ds(u * LANES, LANES)] = zeros_f
                return 0
            lax.fori_loop(0, RANGE_WORDS // NUM_PRODS, zb, 0)

            for s2 in range(NS):
                pltpu.make_async_copy(flat_h.at[pl.ds(0, SEG_CHUNK)],
                                      sorted_flat.at[pl.ds(s2 * SEG_CHUNK, SEG_CHUNK)], sflat).wait()
                pltpu.make_async_copy(amtc_h.at[pl.ds(0, SEG_CHUNK)],
                                      sorted_amt.at[pl.ds(s2 * SEG_CHUNK, SEG_CHUNK)], samt).wait()

            for s2 in range(NS):
                ln = plsc.load_gather(
                    raw_flat, [jnp.full((LANES,), s2 * 2 * NBKT + NBKT + rid, jnp.int32)])[0]
                lnc = jnp.minimum(ln, SEG_CHUNK)
                nv = (lnc + LANES - 1) >> 4

                def sv(j, _):
                    sl = pl.ds(s2 * SEG_CHUNK + j * LANES, LANES)
                    flat = sorted_flat[sl]
                    amtv = sorted_amt[sl]
                    ml = (j * LANES + iota16) < lnc
                    local = flat - base
                    mi = local.astype(jnp.uint32) < jnp.uint32(RANGE_WORDS)
                    m = ml & mi
                    lidx = local & (RANGE_WORDS - 1)
                    rowv = jnp.right_shift(lidx, 8)
                    colv = lidx & (NUM_PRODS - 1)
                    plsc.addupdate_scatter(acc, [rowv, colv], amtv, mask=m)
                    return 0
                lax.fori_loop(0, nv, sv, 0)

                @pl.when(ln > SEG_CHUNK)
                def _slow():
                    st2 = pl.multiple_of(plsc.load_gather(
                        raw_flat, [jnp.full((LANES,), s2 * 2 * NBKT + rid, jnp.int32)])[0], 8)

                    def ch(ci, _):
                        coff = (ci + 1) * SEG_CHUNK
                        src2 = (c * NS + s2) * SORT_CAP + st2 + coff
                        pltpu.sync_copy(flat_h.at[pl.ds(src2, SEG_CHUNK)],
                                        sorted_flat.at[pl.ds(s2 * SEG_CHUNK, SEG_CHUNK)])
                        pltpu.sync_copy(amtc_h.at[pl.ds(src2, SEG_CHUNK)],
                                        sorted_amt.at[pl.ds(s2 * SEG_CHUNK, SEG_CHUNK)])

                        def sv2(j, _):
                            sl = pl.ds(s2 * SEG_CHUNK + j * LANES, LANES)
                            flat = sorted_flat[sl]
                            amtv = sorted_amt[sl]
                            ml = (coff + j * LANES + iota16) < ln
                            local = flat - base
                            mi = local.astype(jnp.uint32) < jnp.uint32(RANGE_WORDS)
                            m = ml & mi
                            lidx = local & (RANGE_WORDS - 1)
                            rowv = jnp.right_shift(lidx, 8)
                            colv = lidx & (NUM_PRODS - 1)
                            plsc.addupdate_scatter(acc, [rowv, colv], amtv, mask=m)
                            return 0
                        lax.fori_loop(0, SEG_CHUNK // LANES, sv2, 0)
                        return 0
                    nch = (ln - 1) >> 9
                    lax.fori_loop(0, nch, ch, 0)

            pltpu.async_copy(
                acc, tot_hbm.at[pl.ds(rid * (RANGE_WORDS // NUM_PRODS),
                                      RANGE_WORDS // NUM_PRODS)], swb)
        return 0
    lax.fori_loop(0, N_PASSES, pass_body, 0)
    pltpu.make_async_copy(
        acc, tot_hbm.at[pl.ds(0, RANGE_WORDS // NUM_PRODS)], swb).wait()


@jax.jit
def _sc_scatter(src, prod, amt):
    mesh = plsc.VectorSubcoreMesh(core_axis_name="c", subcore_axis_name="s")
    return pl.kernel(
        _scatter_kernel,
        out_type=(
            jax.ShapeDtypeStruct((TOT_ROWS, NUM_PRODS), jnp.float32),
            jax.ShapeDtypeStruct((REG_TOTAL,), jnp.int32),
            jax.ShapeDtypeStruct((REG_TOTAL,), jnp.float32),
        ),
        mesh=mesh,
        compiler_params=pltpu.CompilerParams(needs_layout_passes=False),
        scratch_types=[
            pltpu.VMEM((RANGE_WORDS // NUM_PRODS, NUM_PRODS), jnp.float32),
            pltpu.VMEM((PREP_PER_TILE,), jnp.int32),
            pltpu.VMEM((PREP_PER_TILE,), jnp.float32),
            pltpu.VMEM((SORT_CAP,), jnp.int32),
            pltpu.VMEM((SORT_CAP,), jnp.float32),
            pltpu.VMEM((PREP_SUB,), jnp.int32),
            pltpu.VMEM((PREP_SUB,), jnp.int32),
            pltpu.VMEM((PREP_SUB,), jnp.float32),
            pltpu.VMEM((PREP_SUB,), jnp.int32),
            pltpu.VMEM((PREP_SUB,), jnp.int32),
            pltpu.VMEM((PREP_SUB,), jnp.float32),
            pltpu.VMEM((NBKT,), jnp.int32),
            pltpu.VMEM((NBKT,), jnp.int32),
            pltpu.VMEM((NBKT,), jnp.int32),
            pltpu.VMEM((NBKT,), jnp.int32),
            pltpu.VMEM_SHARED((NS * 2 * NBKT,), jnp.int32),
            pltpu.SemaphoreType.DMA,
            pltpu.SemaphoreType.DMA,
            pltpu.SemaphoreType.DMA,
            pltpu.SemaphoreType.DMA,
            pltpu.SemaphoreType.DMA,
        ],
    )(src, prod, amt)


ROWS_PER_BLOCK = 3584
N_BLOCKS = TOT_ROWS // ROWS_PER_BLOCK


def _reduce_kernel(tot_ref, aw_ref, debt_ref, cons_ref):
    i = pl.program_id(0)
    aw = aw_ref[...]
    r = lax.broadcasted_iota(jnp.int32, (NUM_PRODS, NUM_PRODS), 0)
    c = lax.broadcasted_iota(jnp.int32, (NUM_PRODS, NUM_PRODS), 1)
    att = jnp.maximum(jnp.where(r == c, 0.0, aw), 0.0)
    cons = jnp.dot(tot_ref[...], att, preferred_element_type=jnp.float32)
    d = jnp.sum(jnp.maximum(cons - 1.0, 0.0))
    sm = jnp.sum(cons)

    @pl.when(i == 0)
    def _init():
        debt_ref[0, 0] = d
        cons_ref[0, 0] = sm

    @pl.when(i != 0)
    def _acc():
        debt_ref[0, 0] += d
        cons_ref[0, 0] += sm


@jax.jit
def _tc_reduce(totals2d, att_weights):
    return pl.pallas_call(
        _reduce_kernel,
        grid=(N_BLOCKS,),
        in_specs=[
            pl.BlockSpec((ROWS_PER_BLOCK, NUM_PRODS), lambda i: (i, 0)),
            pl.BlockSpec((NUM_PRODS, NUM_PRODS), lambda i: (0, 0)),
        ],
        out_specs=[
            pl.BlockSpec(memory_space=pltpu.SMEM),
            pl.BlockSpec(memory_space=pltpu.SMEM),
        ],
        out_shape=[
            jax.ShapeDtypeStruct((1, 1), jnp.float32),
            jax.ShapeDtypeStruct((1, 1), jnp.float32),
        ],
    )(totals2d, att_weights)


def kernel(src, dst, prod, t, amt, att_weights):
    pad = E_PAD - src.shape[0]
    srcp = jnp.pad(src, (0, pad))
    prodp = jnp.pad(prod, (0, pad))
    amtp = jnp.pad(amt, (0, pad))
    totals2d, _fh, _ah = _sc_scatter(srcp, prodp, amtp)
    debt_sum, cons_sum = _tc_reduce(totals2d, att_weights)
    n = src.shape[0]
    debt_loss = DEBT_PENALTY * debt_sum[0, 0] / NUM_FIRMS
    consump_rwd = CONSUMPTION_REWARD * cons_sum[0, 0] / NUM_FIRMS
    inv_loss = debt_loss - consump_rwd
    return (inv_loss / n, debt_loss / n, consump_rwd / n)


# speedup vs baseline: 7.9086x; 1.0286x over previous
import jax
import jax.numpy as jnp
from jax import lax
from jax.experimental import pallas as pl
from jax.experimental.pallas import tpu as pltpu
from jax.experimental.pallas import tpu_sc as plsc

NUM_FIRMS = 50000
NUM_PRODS = 256
E = 200000
DEBT_PENALTY = 5.0
CONSUMPTION_REWARD = 4.0

NC = 2
NS = 16
NW = NC * NS
LANES = 16

RANGE_WORDS = 65536
N_RANGES = 196
N_PASSES = 7
TOT_ROWS = (N_RANGES * RANGE_WORDS) // NUM_PRODS

E_PAD = 200704
PREP_PER_TILE = E_PAD // NS
PREP_SUB = 1568
N_PREP_SUB = PREP_PER_TILE // PREP_SUB

SORT_CAP = 14336
SEG_CHUNK = 512
REG_TOTAL = NC * NS * SORT_CAP + 2 * SEG_CHUNK
NBKT = 256


def _scatter_kernel(src_hbm, prod_hbm, amt_hbm,
                    tot_hbm, flat_h, amtc_h,
                    acc, raw_flat, raw_amt, sorted_flat, sorted_amt,
                    p_src0, p_prod0, p_amt0, p_src1, p_prod1, p_amt1,
                    cnt, cnt8, off_start, off_run, tbl_sh,
                    sflat, samt, sp0, sp1, swb):
    s = lax.axis_index("s")
    c = lax.axis_index("c")
    wid = s * NC + c
    ones_i = jnp.ones((LANES,), jnp.int32)
    zeros_i = jnp.zeros((LANES,), jnp.int32)
    zeros_f = jnp.zeros((LANES,), jnp.float32)
    neg1 = jnp.full((LANES,), -1, jnp.int32)
    iota16 = lax.iota(jnp.int32, LANES)

    for i in range(NBKT // LANES):
        cnt[pl.ds(i * LANES, LANES)] = zeros_i

    tile_off = s * PREP_PER_TILE

    def issue_prep(k, bs, bp, ba, sem):
        off = tile_off + k * PREP_SUB
        pltpu.async_copy(src_hbm.at[pl.ds(off, PREP_SUB)], bs, sem)
        pltpu.async_copy(prod_hbm.at[pl.ds(off, PREP_SUB)], bp, sem)
        pltpu.async_copy(amt_hbm.at[pl.ds(off, PREP_SUB)], ba, sem)

    def wait_prep(bs, bp, ba, sem):
        pltpu.make_async_copy(src_hbm.at[pl.ds(0, PREP_SUB)], bs, sem).wait()
        pltpu.make_async_copy(prod_hbm.at[pl.ds(0, PREP_SUB)], bp, sem).wait()
        pltpu.make_async_copy(amt_hbm.at[pl.ds(0, PREP_SUB)], ba, sem).wait()

    def compute_prep(k, bs, bp, ba):
        def vb(j, _):
            sl = pl.ds(j * LANES, LANES)
            dst = pl.ds(k * PREP_SUB + j * LANES, LANES)
            flat = bs[sl] * NUM_PRODS + (bp[sl] - NUM_FIRMS)
            raw_flat[dst] = flat
            raw_amt[dst] = jnp.maximum(ba[sl], 0.0)
            ridv = jnp.right_shift(jnp.maximum(flat, 0), 16)
            plsc.addupdate_scatter(cnt, [ridv], ones_i)
            return 0
        lax.fori_loop(0, PREP_SUB // LANES, vb, 0)

    issue_prep(0, p_src0, p_prod0, p_amt0, sp0)
    issue_prep(1, p_src1, p_prod1, p_amt1, sp1)

    def prep_pair(k2, _):
        k0 = k2 * 2
        wait_prep(p_src0, p_prod0, p_amt0, sp0)
        compute_prep(k0, p_src0, p_prod0, p_amt0)

        @pl.when(k2 < N_PREP_SUB // 2 - 1)
        def _i0():
            issue_prep(k0 + 2, p_src0, p_prod0, p_amt0, sp0)
        wait_prep(p_src1, p_prod1, p_amt1, sp1)
        compute_prep(k0 + 1, p_src1, p_prod1, p_amt1)

        @pl.when(k2 < N_PREP_SUB // 2 - 1)
        def _i1():
            issue_prep(k0 + 3, p_src1, p_prod1, p_amt1, sp1)
        return 0
    lax.fori_loop(0, N_PREP_SUB // 2, prep_pair, 0)

    def scan_body(i, carry):
        sl = pl.ds(i * LANES, LANES)
        v8 = (cnt[sl] + 7) & (-8)
        cs = plsc.cumsum(v8)
        offv = cs - v8 + carry
        cnt8[sl] = v8
        off_start[sl] = offv
        off_run[sl] = offv
        return carry + jnp.sum(v8)
    lax.fori_loop(0, NBKT // LANES, scan_body, 0)

    def fill_body(i, _):
        for u in range(4):
            sorted_flat[pl.ds((i * 4 + u) * LANES, LANES)] = neg1
        return 0
    lax.fori_loop(0, SORT_CAP // (4 * LANES), fill_body, 0)

    def place_body(j, _):
        sl = pl.ds(j * LANES, LANES)
        flat = raw_flat[sl]
        amtv = raw_amt[sl]
        ridv = jnp.right_shift(jnp.maximum(flat, 0), 16)
        rank, _ = plsc.scan_count(ridv)
        basev = plsc.load_gather(off_run, [ridv])
        pos = basev + rank - 1
        plsc.store_scatter(sorted_flat, [pos], flat)
        plsc.store_scatter(sorted_amt, [pos], amtv)
        plsc.addupdate_scatter(off_run, [ridv], ones_i)
        return 0
    lax.fori_loop(0, PREP_PER_TILE // LANES, place_body, 0)

    reg_base = (c * NS + s) * SORT_CAP
    pltpu.sync_copy(sorted_flat, flat_h.at[pl.ds(reg_base, SORT_CAP)])
    pltpu.sync_copy(sorted_amt, amtc_h.at[pl.ds(reg_base, SORT_CAP)])
    pltpu.sync_copy(off_start, tbl_sh.at[pl.ds(s * 2 * NBKT, NBKT)])
    pltpu.sync_copy(cnt8, tbl_sh.at[pl.ds(s * 2 * NBKT + NBKT, NBKT)])
    plsc.subcore_barrier()
    pltpu.sync_copy(tbl_sh, raw_flat.at[pl.ds(0, NS * 2 * NBKT)])

    def pass_body(p, _):
        rid = p * NW + wid

        @pl.when(rid < N_RANGES)
        def _active():
            base = rid * RANGE_WORDS

            for s2 in range(NS):
                st = pl.multiple_of(plsc.load_gather(
                    raw_flat, [jnp.full((LANES,), s2 * 2 * NBKT + rid, jnp.int32)])[0], 8)
                src_off = (c * NS + s2) * SORT_CAP + st
                pltpu.async_copy(flat_h.at[pl.ds(src_off, SEG_CHUNK)],
                                 sorted_flat.at[pl.ds(s2 * SEG_CHUNK, SEG_CHUNK)], sflat)
                pltpu.async_copy(amtc_h.at[pl.ds(src_off, SEG_CHUNK)],
                                 sorted_amt.at[pl.ds(s2 * SEG_CHUNK, SEG_CHUNK)], samt)
            @pl.when(p > 0)
            def _drain_wb():
                pltpu.make_async_copy(
                    acc, tot_hbm.at[pl.ds(0, RANGE_WORDS // NUM_PRODS)], swb).wait()

            def zb(i, _):
                for u in range(NUM_PRODS // LANES):
                    acc[i, pl.ds(u * LANES, LANES)] = zeros_f
                return 0
            lax.fori_loop(0, RANGE_WORDS // NUM_PRODS, zb, 0)

            for s2 in range(NS):
                pltpu.make_async_copy(flat_h.at[pl.ds(0, SEG_CHUNK)],
                                      sorted_flat.at[pl.ds(s2 * SEG_CHUNK, SEG_CHUNK)], sflat).wait()
                pltpu.make_async_copy(amtc_h.at[pl.ds(0, SEG_CHUNK)],
                                      sorted_amt.at[pl.ds(s2 * SEG_CHUNK, SEG_CHUNK)], samt).wait()

            for s2 in range(NS):
                ln = plsc.load_gather(
                    raw_flat, [jnp.full((LANES,), s2 * 2 * NBKT + NBKT + rid, jnp.int32)])[0]
                lnc = jnp.minimum(ln, SEG_CHUNK)
                nv = (lnc + LANES - 1) >> 4

                def sv(j, _):
                    sl = pl.ds(s2 * SEG_CHUNK + j * LANES, LANES)
                    flat = sorted_flat[sl]
                    amtv = sorted_amt[sl]
                    ml = (j * LANES + iota16) < lnc
                    local = flat - base
                    mi = local.astype(jnp.uint32) < jnp.uint32(RANGE_WORDS)
                    m = ml & mi
                    lidx = local & (RANGE_WORDS - 1)
                    rowv = jnp.right_shift(lidx, 8)
                    colv = lidx & (NUM_PRODS - 1)
                    plsc.addupdate_scatter(acc, [rowv, colv], amtv, mask=m)
                    return 0
                lax.fori_loop(0, nv, sv, 0)

                @pl.when(ln > SEG_CHUNK)
                def _slow():
                    st2 = pl.multiple_of(plsc.load_gather(
                        raw_flat, [jnp.full((LANES,), s2 * 2 * NBKT + rid, jnp.int32)])[0], 8)

                    def ch(ci, _):
                        coff = (ci + 1) * SEG_CHUNK
                        src2 = (c * NS + s2) * SORT_CAP + st2 + coff
                        pltpu.sync_copy(flat_h.at[pl.ds(src2, SEG_CHUNK)],
                                        sorted_flat.at[pl.ds(s2 * SEG_CHUNK, SEG_CHUNK)])
                        pltpu.sync_copy(amtc_h.at[pl.ds(src2, SEG_CHUNK)],
                                        sorted_amt.at[pl.ds(s2 * SEG_CHUNK, SEG_CHUNK)])

                        def sv2(j, _):
                            sl = pl.ds(s2 * SEG_CHUNK + j * LANES, LANES)
                            flat = sorted_flat[sl]
                            amtv = sorted_amt[sl]
                            ml = (coff + j * LANES + iota16) < ln
                            local = flat - base
                            mi = local.astype(jnp.uint32) < jnp.uint32(RANGE_WORDS)
                            m = ml & mi
                            lidx = local & (RANGE_WORDS - 1)
                            rowv = jnp.right_shift(lidx, 8)
                            colv = lidx & (NUM_PRODS - 1)
                            plsc.addupdate_scatter(acc, [rowv, colv], amtv, mask=m)
                            return 0
                        lax.fori_loop(0, SEG_CHUNK // LANES, sv2, 0)
                        return 0
                    nch = (ln - 1) >> 9
                    lax.fori_loop(0, nch, ch, 0)

            pltpu.async_copy(
                acc, tot_hbm.at[pl.ds(rid * (RANGE_WORDS // NUM_PRODS),
                                      RANGE_WORDS // NUM_PRODS)], swb)
        return 0
    lax.fori_loop(0, N_PASSES, pass_body, 0)
    pltpu.make_async_copy(
        acc, tot_hbm.at[pl.ds(0, RANGE_WORDS // NUM_PRODS)], swb).wait()


@jax.jit
def _sc_scatter(src, prod, amt):
    mesh = plsc.VectorSubcoreMesh(core_axis_name="c", subcore_axis_name="s")
    return pl.kernel(
        _scatter_kernel,
        out_type=(
            jax.ShapeDtypeStruct((TOT_ROWS, NUM_PRODS), jnp.float32),
            jax.ShapeDtypeStruct((REG_TOTAL,), jnp.int32),
            jax.ShapeDtypeStruct((REG_TOTAL,), jnp.float32),
        ),
        mesh=mesh,
        compiler_params=pltpu.CompilerParams(needs_layout_passes=False),
        scratch_types=[
            pltpu.VMEM((RANGE_WORDS // NUM_PRODS, NUM_PRODS), jnp.float32),
            pltpu.VMEM((PREP_PER_TILE,), jnp.int32),
            pltpu.VMEM((PREP_PER_TILE,), jnp.float32),
            pltpu.VMEM((SORT_CAP,), jnp.int32),
            pltpu.VMEM((SORT_CAP,), jnp.float32),
            pltpu.VMEM((PREP_SUB,), jnp.int32),
            pltpu.VMEM((PREP_SUB,), jnp.int32),
            pltpu.VMEM((PREP_SUB,), jnp.float32),
            pltpu.VMEM((PREP_SUB,), jnp.int32),
            pltpu.VMEM((PREP_SUB,), jnp.int32),
            pltpu.VMEM((PREP_SUB,), jnp.float32),
            pltpu.VMEM((NBKT,), jnp.int32),
            pltpu.VMEM((NBKT,), jnp.int32),
            pltpu.VMEM((NBKT,), jnp.int32),
            pltpu.VMEM((NBKT,), jnp.int32),
            pltpu.VMEM_SHARED((NS * 2 * NBKT,), jnp.int32),
            pltpu.SemaphoreType.DMA,
            pltpu.SemaphoreType.DMA,
            pltpu.SemaphoreType.DMA,
            pltpu.SemaphoreType.DMA,
            pltpu.SemaphoreType.DMA,
        ],
    )(src, prod, amt)


ROWS_PER_BLOCK = 7168
N_BLOCKS = TOT_ROWS // ROWS_PER_BLOCK


def _reduce_kernel(tot_ref, aw_ref, debt_ref, cons_ref):
    i = pl.program_id(0)
    aw = aw_ref[...]
    r = lax.broadcasted_iota(jnp.int32, (NUM_PRODS, NUM_PRODS), 0)
    c = lax.broadcasted_iota(jnp.int32, (NUM_PRODS, NUM_PRODS), 1)
    att = jnp.maximum(jnp.where(r == c, 0.0, aw), 0.0)
    cons = jnp.dot(tot_ref[...], att, preferred_element_type=jnp.float32)
    d = jnp.sum(jnp.maximum(cons - 1.0, 0.0))
    sm = jnp.sum(cons)

    @pl.when(i == 0)
    def _init():
        debt_ref[0, 0] = d
        cons_ref[0, 0] = sm

    @pl.when(i != 0)
    def _acc():
        debt_ref[0, 0] += d
        cons_ref[0, 0] += sm


@jax.jit
def _tc_reduce(totals2d, att_weights):
    return pl.pallas_call(
        _reduce_kernel,
        grid=(N_BLOCKS,),
        in_specs=[
            pl.BlockSpec((ROWS_PER_BLOCK, NUM_PRODS), lambda i: (i, 0)),
            pl.BlockSpec((NUM_PRODS, NUM_PRODS), lambda i: (0, 0)),
        ],
        out_specs=[
            pl.BlockSpec(memory_space=pltpu.SMEM),
            pl.BlockSpec(memory_space=pltpu.SMEM),
        ],
        out_shape=[
            jax.ShapeDtypeStruct((1, 1), jnp.float32),
            jax.ShapeDtypeStruct((1, 1), jnp.float32),
        ],
    )(totals2d, att_weights)


def kernel(src, dst, prod, t, amt, att_weights):
    pad = E_PAD - src.shape[0]
    srcp = jnp.pad(src, (0, pad))
    prodp = jnp.pad(prod, (0, pad))
    amtp = jnp.pad(amt, (0, pad))
    totals2d, _fh, _ah = _sc_scatter(srcp, prodp, amtp)
    debt_sum, cons_sum = _tc_reduce(totals2d, att_weights)
    n = src.shape[0]
    debt_loss = DEBT_PENALTY * debt_sum[0, 0] / NUM_FIRMS
    consump_rwd = CONSUMPTION_REWARD * cons_sum[0, 0] / NUM_FIRMS
    inv_loss = debt_loss - consump_rwd
    return (inv_loss / n, debt_loss / n, consump_rwd / n)


# speedup vs baseline: 8.5681x; 1.0834x over previous
import jax
import jax.numpy as jnp
from jax import lax
from jax.experimental import pallas as pl
from jax.experimental.pallas import tpu as pltpu
from jax.experimental.pallas import tpu_sc as plsc

NUM_FIRMS = 50000
NUM_PRODS = 256
E = 200000
DEBT_PENALTY = 5.0
CONSUMPTION_REWARD = 4.0

NC = 2
NS = 16
NW = NC * NS
LANES = 16

RANGE_WORDS = 32768
RANGE_SHIFT = 15
RANGE_ROWS = RANGE_WORDS // NUM_PRODS
N_RANGES = 392
N_PASSES = 13
TOT_ROWS = (N_RANGES * RANGE_WORDS) // NUM_PRODS

E_PAD = 200704
PREP_PER_TILE = E_PAD // NS
PREP_SUB = 784
N_PREP_SUB = PREP_PER_TILE // PREP_SUB

SORT_CAP = 15360
SEG_CHUNK = 256
REG_TOTAL = NC * NS * SORT_CAP + 2 * SEG_CHUNK
NBKT = 512
SEG_BUF_OFF = 0
CNT_TBL_OFF = 6144


def _scatter_kernel(src_hbm, prod_hbm, amt_hbm,
                    tot_hbm, flat_h, amtc_h,
                    acc0, acc1, raw_flat, raw_amt, sorted_flat, sorted_amt,
                    p_src0, p_prod0, p_amt0, p_src1, p_prod1, p_amt1,
                    cnt, cnt8, off_start, off_run, tbl_off_sh, tbl_cnt_sh,
                    sflat, samt, sp0, sp1, swb0, swb1):
    s = lax.axis_index("s")
    c = lax.axis_index("c")
    wid = s * NC + c
    ones_i = jnp.ones((LANES,), jnp.int32)
    zeros_i = jnp.zeros((LANES,), jnp.int32)
    zeros_f = jnp.zeros((LANES,), jnp.float32)
    neg1 = jnp.full((LANES,), -1, jnp.int32)
    iota16 = lax.iota(jnp.int32, LANES)

    for i in range(NBKT // LANES):
        cnt[pl.ds(i * LANES, LANES)] = zeros_i

    tile_off = s * PREP_PER_TILE

    def issue_prep(k, bs, bp, ba, sem):
        off = tile_off + k * PREP_SUB
        pltpu.async_copy(src_hbm.at[pl.ds(off, PREP_SUB)], bs, sem)
        pltpu.async_copy(prod_hbm.at[pl.ds(off, PREP_SUB)], bp, sem)
        pltpu.async_copy(amt_hbm.at[pl.ds(off, PREP_SUB)], ba, sem)

    def wait_prep(bs, bp, ba, sem):
        pltpu.make_async_copy(src_hbm.at[pl.ds(0, PREP_SUB)], bs, sem).wait()
        pltpu.make_async_copy(prod_hbm.at[pl.ds(0, PREP_SUB)], bp, sem).wait()
        pltpu.make_async_copy(amt_hbm.at[pl.ds(0, PREP_SUB)], ba, sem).wait()

    def compute_prep(k, bs, bp, ba):
        def vb(j, _):
            sl = pl.ds(j * LANES, LANES)
            dst = pl.ds(k * PREP_SUB + j * LANES, LANES)
            flat = bs[sl] * NUM_PRODS + (bp[sl] - NUM_FIRMS)
            raw_flat[dst] = flat
            raw_amt[dst] = jnp.maximum(ba[sl], 0.0)
            ridv = jnp.right_shift(jnp.maximum(flat, 0), RANGE_SHIFT)
            plsc.addupdate_scatter(cnt, [ridv], ones_i)
            return 0
        lax.fori_loop(0, PREP_SUB // LANES, vb, 0)

    issue_prep(0, p_src0, p_prod0, p_amt0, sp0)
    issue_prep(1, p_src1, p_prod1, p_amt1, sp1)

    def prep_pair(k2, _):
        k0 = k2 * 2
        wait_prep(p_src0, p_prod0, p_amt0, sp0)
        compute_prep(k0, p_src0, p_prod0, p_amt0)

        @pl.when(k2 < N_PREP_SUB // 2 - 1)
        def _i0():
            issue_prep(k0 + 2, p_src0, p_prod0, p_amt0, sp0)
        wait_prep(p_src1, p_prod1, p_amt1, sp1)
        compute_prep(k0 + 1, p_src1, p_prod1, p_amt1)

        @pl.when(k2 < N_PREP_SUB // 2 - 1)
        def _i1():
            issue_prep(k0 + 3, p_src1, p_prod1, p_amt1, sp1)
        return 0
    lax.fori_loop(0, N_PREP_SUB // 2, prep_pair, 0)

    def scan_body(i, carry):
        sl = pl.ds(i * LANES, LANES)
        v8 = (cnt[sl] + 7) & (-8)
        cs = plsc.cumsum(v8)
        offv = cs - v8 + carry
        cnt8[sl] = v8
        off_start[sl] = offv
        off_run[sl] = offv
        return carry + jnp.sum(v8)
    lax.fori_loop(0, NBKT // LANES, scan_body, 0)

    def fill_body(i, _):
        for u in range(4):
            sorted_flat[pl.ds((i * 4 + u) * LANES, LANES)] = neg1
        return 0
    lax.fori_loop(0, SORT_CAP // (4 * LANES), fill_body, 0)

    def place_body(j, _):
        sl = pl.ds(j * LANES, LANES)
        flat = raw_flat[sl]
        amtv = raw_amt[sl]
        ridv = jnp.right_shift(jnp.maximum(flat, 0), RANGE_SHIFT)
        rank, _ = plsc.scan_count(ridv)
        basev = plsc.load_gather(off_run, [ridv])
        pos = basev + rank - 1
        plsc.store_scatter(sorted_flat, [pos], flat)
        plsc.store_scatter(sorted_amt, [pos], amtv)
        plsc.addupdate_scatter(off_run, [ridv], ones_i)
        return 0
    lax.fori_loop(0, PREP_PER_TILE // LANES, place_body, 0)

    reg_base = (c * NS + s) * SORT_CAP
    pltpu.sync_copy(sorted_flat, flat_h.at[pl.ds(reg_base, SORT_CAP)])
    pltpu.sync_copy(sorted_amt, amtc_h.at[pl.ds(reg_base, SORT_CAP)])
    pltpu.sync_copy(off_start, tbl_off_sh.at[pl.ds(s * NBKT, NBKT)])
    pltpu.sync_copy(cnt8, tbl_cnt_sh.at[pl.ds(s * NBKT, NBKT)])
    plsc.subcore_barrier()
    pltpu.sync_copy(tbl_off_sh, raw_flat.at[pl.ds(0, NS * NBKT)])
    pltpu.sync_copy(tbl_cnt_sh, sorted_flat.at[pl.ds(CNT_TBL_OFF, NS * NBKT)])

    def do_pass(p, acc, swb, first_use):
        rid = p * NW + wid

        @pl.when(rid < N_RANGES)
        def _active():
            base = rid * RANGE_WORDS

            def fetch_seg(s2, _):
                st = pl.multiple_of(plsc.load_gather(
                    raw_flat, [jnp.full((LANES,), s2 * NBKT + rid, jnp.int32)])[0], 8)
                src_off = (c * NS + s2) * SORT_CAP + st
                pltpu.async_copy(flat_h.at[pl.ds(src_off, SEG_CHUNK)],
                                 sorted_flat.at[pl.ds(SEG_BUF_OFF + s2 * SEG_CHUNK, SEG_CHUNK)], sflat)
                pltpu.async_copy(amtc_h.at[pl.ds(src_off, SEG_CHUNK)],
                                 sorted_amt.at[pl.ds(SEG_BUF_OFF + s2 * SEG_CHUNK, SEG_CHUNK)], samt)
                return 0
            lax.fori_loop(0, NS, fetch_seg, 0)

            if not first_use:
                pltpu.make_async_copy(
                    acc, tot_hbm.at[pl.ds(0, RANGE_ROWS)], swb).wait()

            def zb(i, _):
                for u in range(NUM_PRODS // LANES):
                    acc[i, pl.ds(u * LANES, LANES)] = zeros_f
                return 0
            lax.fori_loop(0, RANGE_ROWS, zb, 0)

            def wait_seg(s2, _):
                pltpu.make_async_copy(flat_h.at[pl.ds(0, SEG_CHUNK)],
                                      sorted_flat.at[pl.ds(SEG_BUF_OFF + s2 * SEG_CHUNK, SEG_CHUNK)], sflat).wait()
                pltpu.make_async_copy(amtc_h.at[pl.ds(0, SEG_CHUNK)],
                                      sorted_amt.at[pl.ds(SEG_BUF_OFF + s2 * SEG_CHUNK, SEG_CHUNK)], samt).wait()
                return 0
            lax.fori_loop(0, NS, wait_seg, 0)

            def scan_seg(s2, _):
                ln = plsc.load_gather(
                    sorted_flat, [jnp.full((LANES,), CNT_TBL_OFF + s2 * NBKT + rid, jnp.int32)])[0]
                lnc = jnp.minimum(ln, SEG_CHUNK)
                nv = (lnc + LANES - 1) >> 4

                def sv(j, _):
                    sl = pl.ds(SEG_BUF_OFF + s2 * SEG_CHUNK + j * LANES, LANES)
                    flat = sorted_flat[sl]
                    amtv = sorted_amt[sl]
                    ml = (j * LANES + iota16) < lnc
                    local = flat - base
                    mi = local.astype(jnp.uint32) < jnp.uint32(RANGE_WORDS)
                    m = ml & mi
                    lidx = local & (RANGE_WORDS - 1)
                    rowv = jnp.right_shift(lidx, 8)
                    colv = lidx & (NUM_PRODS - 1)
                    plsc.addupdate_scatter(acc, [rowv, colv], amtv, mask=m)
                    return 0
                lax.fori_loop(0, nv, sv, 0)

                @pl.when(ln > SEG_CHUNK)
                def _slow():
                    st2 = pl.multiple_of(plsc.load_gather(
                        raw_flat, [jnp.full((LANES,), s2 * NBKT + rid, jnp.int32)])[0], 8)

                    def ch(ci, _):
                        coff = (ci + 1) * SEG_CHUNK
                        src2 = (c * NS + s2) * SORT_CAP + st2 + coff
                        pltpu.sync_copy(flat_h.at[pl.ds(src2, SEG_CHUNK)],
                                        sorted_flat.at[pl.ds(SEG_BUF_OFF + s2 * SEG_CHUNK, SEG_CHUNK)])
                        pltpu.sync_copy(amtc_h.at[pl.ds(src2, SEG_CHUNK)],
                                        sorted_amt.at[pl.ds(SEG_BUF_OFF + s2 * SEG_CHUNK, SEG_CHUNK)])

                        def sv2(j, _):
                            sl = pl.ds(SEG_BUF_OFF + s2 * SEG_CHUNK + j * LANES, LANES)
                            flat = sorted_flat[sl]
                            amtv = sorted_amt[sl]
                            ml = (coff + j * LANES + iota16) < ln
                            local = flat - base
                            mi = local.astype(jnp.uint32) < jnp.uint32(RANGE_WORDS)
                            m = ml & mi
                            lidx = local & (RANGE_WORDS - 1)
                            rowv = jnp.right_shift(lidx, 8)
                            colv = lidx & (NUM_PRODS - 1)
                            plsc.addupdate_scatter(acc, [rowv, colv], amtv, mask=m)
                            return 0
                        lax.fori_loop(0, SEG_CHUNK // LANES, sv2, 0)
                        return 0
                    nch = (ln - 1) >> 8
                    lax.fori_loop(0, nch, ch, 0)
                return 0
            lax.fori_loop(0, NS, scan_seg, 0)

            pltpu.async_copy(
                acc, tot_hbm.at[pl.ds(rid * RANGE_ROWS, RANGE_ROWS)], swb)

    do_pass(0, acc0, swb0, True)
    do_pass(1, acc1, swb1, True)

    def pass_pair(q, _):
        do_pass(q * 2, acc0, swb0, False)
        do_pass(q * 2 + 1, acc1, swb1, False)
        return 0
    lax.fori_loop(1, 6, pass_pair, 0)
    do_pass(12, acc0, swb0, False)

    pltpu.make_async_copy(acc0, tot_hbm.at[pl.ds(0, RANGE_ROWS)], swb0).wait()
    pltpu.make_async_copy(acc1, tot_hbm.at[pl.ds(0, RANGE_ROWS)], swb1).wait()


@jax.jit
def _sc_scatter(src, prod, amt):
    mesh = plsc.VectorSubcoreMesh(core_axis_name="c", subcore_axis_name="s")
    return pl.kernel(
        _scatter_kernel,
        out_type=(
            jax.ShapeDtypeStruct((TOT_ROWS, NUM_PRODS), jnp.float32),
            jax.ShapeDtypeStruct((REG_TOTAL,), jnp.int32),
            jax.ShapeDtypeStruct((REG_TOTAL,), jnp.float32),
        ),
        mesh=mesh,
        compiler_params=pltpu.CompilerParams(needs_layout_passes=False),
        scratch_types=[
            pltpu.VMEM((RANGE_ROWS, NUM_PRODS), jnp.float32),
            pltpu.VMEM((RANGE_ROWS, NUM_PRODS), jnp.float32),
            pltpu.VMEM((PREP_PER_TILE,), jnp.int32),
            pltpu.VMEM((PREP_PER_TILE,), jnp.float32),
            pltpu.VMEM((SORT_CAP,), jnp.int32),
            pltpu.VMEM((SORT_CAP,), jnp.float32),
            pltpu.VMEM((PREP_SUB,), jnp.int32),
            pltpu.VMEM((PREP_SUB,), jnp.int32),
            pltpu.VMEM((PREP_SUB,), jnp.float32),
            pltpu.VMEM((PREP_SUB,), jnp.int32),
            pltpu.VMEM((PREP_SUB,), jnp.int32),
            pltpu.VMEM((PREP_SUB,), jnp.float32),
            pltpu.VMEM((NBKT,), jnp.int32),
            pltpu.VMEM((NBKT,), jnp.int32),
            pltpu.VMEM((NBKT,), jnp.int32),
            pltpu.VMEM((NBKT,), jnp.int32),
            pltpu.VMEM_SHARED((NS * NBKT,), jnp.int32),
            pltpu.VMEM_SHARED((NS * NBKT,), jnp.int32),
            pltpu.SemaphoreType.DMA,
            pltpu.SemaphoreType.DMA,
            pltpu.SemaphoreType.DMA,
            pltpu.SemaphoreType.DMA,
            pltpu.SemaphoreType.DMA,
            pltpu.SemaphoreType.DMA,
        ],
    )(src, prod, amt)


ROWS_PER_BLOCK = 7168
N_BLOCKS = TOT_ROWS // ROWS_PER_BLOCK


def _reduce_kernel(tot_ref, aw_ref, debt_ref, cons_ref):
    i = pl.program_id(0)
    aw = aw_ref[...]
    r = lax.broadcasted_iota(jnp.int32, (NUM_PRODS, NUM_PRODS), 0)
    c = lax.broadcasted_iota(jnp.int32, (NUM_PRODS, NUM_PRODS), 1)
    att = jnp.maximum(jnp.where(r == c, 0.0, aw), 0.0)
    cons = jnp.dot(tot_ref[...], att, preferred_element_type=jnp.float32)
    d = jnp.sum(jnp.maximum(cons - 1.0, 0.0))
    sm = jnp.sum(cons)

    @pl.when(i == 0)
    def _init():
        debt_ref[0, 0] = d
        cons_ref[0, 0] = sm

    @pl.when(i != 0)
    def _acc():
        debt_ref[0, 0] += d
        cons_ref[0, 0] += sm


@jax.jit
def _tc_reduce(totals2d, att_weights):
    return pl.pallas_call(
        _reduce_kernel,
        grid=(N_BLOCKS,),
        in_specs=[
            pl.BlockSpec((ROWS_PER_BLOCK, NUM_PRODS), lambda i: (i, 0)),
            pl.BlockSpec((NUM_PRODS, NUM_PRODS), lambda i: (0, 0)),
        ],
        out_specs=[
            pl.BlockSpec(memory_space=pltpu.SMEM),
            pl.BlockSpec(memory_space=pltpu.SMEM),
        ],
        out_shape=[
            jax.ShapeDtypeStruct((1, 1), jnp.float32),
            jax.ShapeDtypeStruct((1, 1), jnp.float32),
        ],
    )(totals2d, att_weights)


def kernel(src, dst, prod, t, amt, att_weights):
    pad = E_PAD - src.shape[0]
    srcp = jnp.pad(src, (0, pad))
    prodp = jnp.pad(prod, (0, pad))
    amtp = jnp.pad(amt, (0, pad))
    totals2d, _fh, _ah = _sc_scatter(srcp, prodp, amtp)
    debt_sum, cons_sum = _tc_reduce(totals2d, att_weights)
    n = src.shape[0]
    debt_loss = DEBT_PENALTY * debt_sum[0, 0] / NUM_FIRMS
    consump_rwd = CONSUMPTION_REWARD * cons_sum[0, 0] / NUM_FIRMS
    inv_loss = debt_loss - consump_rwd
    return (inv_loss / n, debt_loss / n, consump_rwd / n)
